# Initial kernel scaffold; baseline (speedup 1.0000x reference)
import jax
import jax.numpy as jnp
from jax.experimental import pallas as pl


def kernel(params, constraint_features, edge_indices, edge_features, variable_features):
    raise NotImplementedError("write your pallas kernel here")


# speedup vs baseline: 1.4405x; 1.4405x over previous
import functools

import jax
import jax.numpy as jnp
from jax import lax
from jax.experimental import pallas as pl
from jax.experimental.pallas import tpu as pltpu
from jax.experimental.pallas import tpu_sc as plsc

F32 = jnp.float32
EMBD = 64
PK = 2 * EMBD
NBLK = 1000
EBLK = 2000
EPS = 1e-5

SC_CORES = 2
SC_SUBCORES = 16
NWORK = SC_CORES * SC_SUBCORES
GCHUNK = 128


def _mm(x, w):
    return lax.dot_general(x, w, (((1,), (1,)), ((), ())),
                           preferred_element_type=F32)


def _lnf(x, g, b):
    m = jnp.mean(x, axis=-1, keepdims=True)
    v = jnp.mean((x - m) ** 2, axis=-1, keepdims=True)
    return (x - m) / jnp.sqrt(v + EPS) * g + b


def _bspec(shape, imap):
    return pl.BlockSpec(shape, imap)


def _embed_body(xa_ref, xb_ref, g_ref, b_ref, w1_ref, b1_ref, w2_ref, b2_ref,
                out_ref, cs_ref):
    def half(x):
        h = _lnf(x, g_ref[...], b_ref[...])
        h = jnp.maximum(_mm(h, w1_ref[...]) + b1_ref[...], 0.0)
        return jnp.maximum(_mm(h, w2_ref[...]) + b2_ref[...], 0.0)

    ha = half(xa_ref[...])
    hb = half(xb_ref[...])
    out_ref[:, :EMBD] = ha
    out_ref[:, EMBD:] = hb

    @pl.when(pl.program_id(0) == 0)
    def _():
        cs_ref[...] = jnp.zeros_like(cs_ref)

    cs_ref[:, :EMBD] += jnp.sum(ha, axis=0, keepdims=True)
    cs_ref[:, EMBD:] += jnp.sum(hb, axis=0, keepdims=True)


def _embed(x, g, b, w1, b1, w2, b2):
    n, fin = x.shape
    n2 = n // 2
    grid = n2 // NBLK
    out, cs = pl.pallas_call(
        _embed_body,
        grid=(grid,),
        in_specs=[
            _bspec((NBLK, fin), lambda i: (i, 0)),
            _bspec((NBLK, fin), lambda i, g=grid: (i + g, 0)),
            _bspec((1, fin), lambda i: (0, 0)),
            _bspec((1, fin), lambda i: (0, 0)),
            _bspec((EMBD, fin), lambda i: (0, 0)),
            _bspec((1, EMBD), lambda i: (0, 0)),
            _bspec((EMBD, EMBD), lambda i: (0, 0)),
            _bspec((1, EMBD), lambda i: (0, 0)),
        ],
        out_specs=[
            _bspec((NBLK, PK), lambda i: (i, 0)),
            _bspec((1, PK), lambda i: (0, 0)),
        ],
        out_shape=[
            jax.ShapeDtypeStruct((n2, PK), F32),
            jax.ShapeDtypeStruct((1, PK), F32),
        ],
        compiler_params=pltpu.CompilerParams(
            dimension_semantics=("arbitrary",)),
    )(x, x, g.reshape(1, fin), b.reshape(1, fin), w1, b1.reshape(1, EMBD),
      w2, b2.reshape(1, EMBD))
    return out, cs


def _se_body(x_ref, cs_ref, w1_ref, w2_ref, out_ref, *, inv_n):
    mean = (cs_ref[:, :EMBD] + cs_ref[:, EMBD:]) * inv_n
    h = jnp.maximum(_mm(mean, w1_ref[...]), 0.0)
    w = jax.nn.sigmoid(_mm(h, w2_ref[...]))
    out_ref[:, :EMBD] = x_ref[:, :EMBD] * w
    out_ref[:, EMBD:] = x_ref[:, EMBD:] * w


def _se_apply(x, cs, w1, w2):
    n2 = x.shape[0]
    sq = w1.shape[0]
    return pl.pallas_call(
        functools.partial(_se_body, inv_n=0.5 / n2),
        grid=(n2 // NBLK,),
        in_specs=[
            _bspec((NBLK, PK), lambda i: (i, 0)),
            _bspec((1, PK), lambda i: (0, 0)),
            _bspec((sq, EMBD), lambda i: (0, 0)),
            _bspec((EMBD, sq), lambda i: (0, 0)),
        ],
        out_specs=_bspec((NBLK, PK), lambda i: (i, 0)),
        out_shape=jax.ShapeDtypeStruct((n2, PK), F32),
    )(x, cs, w1, w2)


def _pq_body(r_ref, l_ref, wl_ref, blc_ref, wr_ref, p_ref, q_ref):
    p_ref[:, :EMBD] = _mm(r_ref[:, :EMBD], wl_ref[...]) + blc_ref[...]
    p_ref[:, EMBD:] = _mm(r_ref[:, EMBD:], wl_ref[...]) + blc_ref[...]
    q_ref[:, :EMBD] = _mm(l_ref[:, :EMBD], wr_ref[...])
    q_ref[:, EMBD:] = _mm(l_ref[:, EMBD:], wr_ref[...])


def _pq(right, left, wl, blc, wr):
    n2 = right.shape[0]
    return pl.pallas_call(
        _pq_body,
        grid=(n2 // NBLK,),
        in_specs=[
            _bspec((NBLK, PK), lambda i: (i, 0)),
            _bspec((NBLK, PK), lambda i: (i, 0)),
            _bspec((EMBD, EMBD), lambda i: (0, 0)),
            _bspec((1, EMBD), lambda i: (0, 0)),
            _bspec((EMBD, EMBD), lambda i: (0, 0)),
        ],
        out_specs=[
            _bspec((NBLK, PK), lambda i: (i, 0)),
            _bspec((NBLK, PK), lambda i: (i, 0)),
        ],
        out_shape=[
            jax.ShapeDtypeStruct((n2, PK), F32),
            jax.ShapeDtypeStruct((n2, PK), F32),
        ],
    )(right, left, wl, blc, wr)


def _edge_body(pg_ref, qg_ref, d_ref, s_ref, g_ref, b_ref, wf_ref, bf_ref,
               msg_ref, *, n_half):
    pd = d_ref[...] >= n_half
    ps = s_ref[...] >= n_half
    p = jnp.where(pd, pg_ref[:, EMBD:], pg_ref[:, :EMBD])
    q = jnp.where(ps, qg_ref[:, EMBD:], qg_ref[:, :EMBD])
    h = _lnf(p + q, g_ref[...], b_ref[...])
    h = jnp.maximum(h, 0.0)
    m = _mm(h, wf_ref[...]) + bf_ref[...]
    msg_ref[:, :EMBD] = jnp.where(pd, 0.0, m)
    msg_ref[:, EMBD:] = jnp.where(pd, m, 0.0)


def _edge_mlp(pg, qg, d2, s2, g, b, wf, bf, n_half):
    e = pg.shape[0]
    return pl.pallas_call(
        functools.partial(_edge_body, n_half=n_half),
        grid=(e // EBLK,),
        in_specs=[
            _bspec((EBLK, PK), lambda i: (i, 0)),
            _bspec((EBLK, PK), lambda i: (i, 0)),
            _bspec((EBLK, 1), lambda i: (i, 0)),
            _bspec((EBLK, 1), lambda i: (i, 0)),
            _bspec((1, EMBD), lambda i: (0, 0)),
            _bspec((1, EMBD), lambda i: (0, 0)),
            _bspec((EMBD, EMBD), lambda i: (0, 0)),
            _bspec((1, EMBD), lambda i: (0, 0)),
        ],
        out_specs=_bspec((EBLK, PK), lambda i: (i, 0)),
        out_shape=jax.ShapeDtypeStruct((e, PK), F32),
    )(pg, qg, d2, s2, g.reshape(1, EMBD), b.reshape(1, EMBD), wf,
      bf.reshape(1, EMBD))


def _post_body(agg_ref, r_ref, g_ref, b_ref, w1a_ref, w1b_ref, bo1_ref,
               w2_ref, bo2_ref, out_ref):
    def half(sl):
        post = _lnf(agg_ref[:, sl], g_ref[...], b_ref[...])
        h = (_mm(post, w1a_ref[...]) + _mm(r_ref[:, sl], w1b_ref[...])
             + bo1_ref[...])
        h = jnp.maximum(h, 0.0)
        return _mm(h, w2_ref[...]) + bo2_ref[...]

    out_ref[:, :EMBD] = half(slice(0, EMBD))
    out_ref[:, EMBD:] = half(slice(EMBD, PK))


def _post(agg, right, g, b, w1a, w1b, bo1, w2, bo2):
    n2 = agg.shape[0]
    return pl.pallas_call(
        _post_body,
        grid=(n2 // NBLK,),
        in_specs=[
            _bspec((NBLK, PK), lambda i: (i, 0)),
            _bspec((NBLK, PK), lambda i: (i, 0)),
            _bspec((1, EMBD), lambda i: (0, 0)),
            _bspec((1, EMBD), lambda i: (0, 0)),
            _bspec((EMBD, EMBD), lambda i: (0, 0)),
            _bspec((EMBD, EMBD), lambda i: (0, 0)),
            _bspec((1, EMBD), lambda i: (0, 0)),
            _bspec((EMBD, EMBD), lambda i: (0, 0)),
            _bspec((1, EMBD), lambda i: (0, 0)),
        ],
        out_specs=_bspec((NBLK, PK), lambda i: (i, 0)),
        out_shape=jax.ShapeDtypeStruct((n2, PK), F32),
    )(agg, right, g.reshape(1, EMBD), b.reshape(1, EMBD), w1a, w1b,
      bo1.reshape(1, EMBD), w2, bo2.reshape(1, EMBD))


def _head_body(x_ref, w1_ref, b1_ref, w2_ref, oa_ref, ob_ref):
    def half(sl):
        h = jnp.maximum(_mm(x_ref[:, sl], w1_ref[...]) + b1_ref[...], 0.0)
        return jax.nn.sigmoid(_mm(h, w2_ref[...]) * (1.0 / 0.6))

    oa_ref[...] = half(slice(0, EMBD))
    ob_ref[...] = half(slice(EMBD, PK))


def _head(x, w1, b1, w2):
    n2 = x.shape[0]
    oa, ob = pl.pallas_call(
        _head_body,
        grid=(n2 // NBLK,),
        in_specs=[
            _bspec((NBLK, PK), lambda i: (i, 0)),
            _bspec((EMBD, EMBD), lambda i: (0, 0)),
            _bspec((1, EMBD), lambda i: (0, 0)),
            _bspec((1, EMBD), lambda i: (0, 0)),
        ],
        out_specs=[
            _bspec((NBLK, 1), lambda i: (i, 0)),
            _bspec((NBLK, 1), lambda i: (i, 0)),
        ],
        out_shape=[
            jax.ShapeDtypeStruct((n2, 1), F32),
            jax.ShapeDtypeStruct((n2, 1), F32),
        ],
    )(x, w1, b1.reshape(1, EMBD), w2)
    return jnp.concatenate([oa, ob], axis=0)


def _sc_gather(p_tab, q_tab, d_idx, s_idx):
    e = d_idx.shape[0]
    n_half = p_tab.shape[0]
    per_w = e // NWORK
    iters = per_w // GCHUNK
    tail = per_w - iters * GCHUNK
    mesh = plsc.VectorSubcoreMesh(core_axis_name="c", subcore_axis_name="s")

    @functools.partial(
        pl.kernel, mesh=mesh,
        out_type=(jax.ShapeDtypeStruct((e, PK), F32),
                  jax.ShapeDtypeStruct((e, PK), F32)),
        scratch_types=[
            pltpu.VMEM((GCHUNK,), jnp.int32),
            pltpu.VMEM((GCHUNK,), jnp.int32),
            pltpu.VMEM((GCHUNK, PK), F32),
            pltpu.VMEM((GCHUNK, PK), F32),
            pltpu.VMEM((tail,), jnp.int32),
            pltpu.VMEM((tail,), jnp.int32),
            pltpu.VMEM((tail, PK), F32),
            pltpu.VMEM((tail, PK), F32),
            pltpu.SemaphoreType.DMA,
            pltpu.SemaphoreType.DMA,
        ],
    )
    def k(p_hbm, q_hbm, d_hbm, s_hbm, pg_hbm, qg_hbm,
          di, si, pr, qr, dit, sit, prt, qrt, sem1, sem2):
        wid = lax.axis_index("s") * SC_CORES + lax.axis_index("c")
        base = wid * per_w

        def physify(ref, count):
            for kk in range(count // 16):
                v = ref[pl.ds(kk * 16, 16)]
                ref[pl.ds(kk * 16, 16)] = jnp.where(v >= n_half, v - n_half, v)

        def step(off, dref, sref, prref, qrref, count):
            pltpu.sync_copy(d_hbm.at[pl.ds(off, count)], dref)
            pltpu.sync_copy(s_hbm.at[pl.ds(off, count)], sref)
            physify(dref, count)
            physify(sref, count)
            c1 = pltpu.async_copy(p_hbm.at[dref], prref, sem1)
            c2 = pltpu.async_copy(q_hbm.at[sref], qrref, sem2)
            c1.wait()
            c2.wait()
            pltpu.sync_copy(prref, pg_hbm.at[pl.ds(off, count)])
            pltpu.sync_copy(qrref, qg_hbm.at[pl.ds(off, count)])

        def body(j, carry):
            step(base + j * GCHUNK, di, si, pr, qr, GCHUNK)
            return carry

        lax.fori_loop(0, iters, body, 0)
        if tail:
            step(base + iters * GCHUNK, dit, sit, prt, qrt, tail)

    return k(p_tab, q_tab, d_idx, s_idx)


def _sc_scatter(msg, d_idx, zeros, n_half):
    e = d_idx.shape[0]
    half = n_half // SC_CORES
    pad = zeros.shape[0]
    rows_pt = pad // SC_SUBCORES
    per_t = e // SC_SUBCORES
    iters = per_t // GCHUNK
    tail = per_t - iters * GCHUNK
    mesh = plsc.VectorSubcoreMesh(core_axis_name="c", subcore_axis_name="s")

    @functools.partial(
        pl.kernel, mesh=mesh,
        out_type=jax.ShapeDtypeStruct((SC_CORES, pad, PK), F32),
        scratch_types=[
            pltpu.VMEM((GCHUNK,), jnp.int32),
            pltpu.VMEM((GCHUNK, PK), F32),
            pltpu.VMEM((tail,), jnp.int32),
            pltpu.VMEM((tail, PK), F32),
            pltpu.VMEM_SHARED((pad, PK), F32),
        ],
    )
    def k(msg_hbm, d_hbm, z_hbm, out_hbm, draw, mbuf, drawt, mbuft, acc):
        cid = lax.axis_index("c")
        sid = lax.axis_index("s")
        base = cid * half
        dump = half + sid

        zr = sid * rows_pt
        pltpu.sync_copy(z_hbm.at[pl.ds(zr, rows_pt)],
                        acc.at[pl.ds(zr, rows_pt)])
        plsc.subcore_barrier()

        def localize(ref, count):
            for kk in range(count // 16):
                v = ref[pl.ds(kk * 16, 16)]
                v = jnp.where(v >= n_half, v - n_half, v)
                l = v - base
                ok = (l >= 0) & (l < half)
                ref[pl.ds(kk * 16, 16)] = jnp.where(ok, l, dump)

        def step(off, dref, mref, count):
            pltpu.sync_copy(d_hbm.at[pl.ds(off, count)], dref)
            pltpu.sync_copy(msg_hbm.at[pl.ds(off, count)], mref)
            localize(dref, count)
            pltpu.sync_copy(mref, acc.at[dref], add=True)

        def body(j, carry):
            step(sid * per_t + j * GCHUNK, draw, mbuf, GCHUNK)
            return carry

        lax.fori_loop(0, iters, body, 0)
        if tail:
            step(sid * per_t + iters * GCHUNK, drawt, mbuft, tail)

        plsc.subcore_barrier()
        pltpu.sync_copy(acc.at[pl.ds(zr, rows_pt)],
                        out_hbm.at[cid, pl.ds(zr, rows_pt)])

    out = k(msg, d_idx, zeros)
    return jnp.concatenate([out[0, :half], out[1, :half]], axis=0)


def _conv(cp, left, right, s_idx, d_idx, s2, d2, edge_b, zeros):
    n_half = right.shape[0]
    blc = (cp['bl'] + edge_b * cp['We'][:, 0]).reshape(1, EMBD)
    p_tab, q_tab = _pq(right, left, cp['Wl'], blc, cp['Wr'])
    pg, qg = _sc_gather(p_tab, q_tab, d_idx, s_idx)
    msg = _edge_mlp(pg, qg, d2, s2, cp['ln_g'], cp['ln_b'], cp['Wf'],
                    cp['bf'], n_half)
    agg = _sc_scatter(msg, d_idx, zeros, n_half)
    return _post(agg, right, cp['pc_g'], cp['pc_b'],
                 cp['Wo1'][:, :EMBD], cp['Wo1'][:, EMBD:], cp['bo1'],
                 cp['Wo2'], cp['bo2'])


def kernel(params, constraint_features, edge_indices, edge_features,
           variable_features):
    p = params
    src = edge_indices[0]
    dst = edge_indices[1]
    e = src.shape[0]
    src2 = src.reshape(e, 1)
    dst2 = dst.reshape(e, 1)
    n_c = constraint_features.shape[0]
    edge_b = p['edge_ln_b'][0]

    half = n_c // 2 // SC_CORES
    pad = ((half + SC_SUBCORES) + 127) // 128 * 128
    zeros = jnp.zeros((pad, PK), F32)

    cf, cs = _embed(constraint_features, p['cons_ln_g'], p['cons_ln_b'],
                    p['cons_W1'], p['cons_b1'], p['cons_W2'], p['cons_b2'])
    cf = _se_apply(cf, cs, p['se_con_W1'], p['se_con_W2'])
    vf, vs = _embed(variable_features, p['var_ln_g'], p['var_ln_b'],
                    p['var_W1'], p['var_b1'], p['var_W2'], p['var_b2'])
    vf = _se_apply(vf, vs, p['se_var_W1'], p['se_var_W2'])

    vf = _conv(p['c2v'], cf, vf, src, dst, src2, dst2, edge_b, zeros)
    cf = _conv(p['v2c'], vf, cf, dst, src, dst2, src2, edge_b, zeros)
    vf = _conv(p['c2v2'], cf, vf, src, dst, src2, dst2, edge_b, zeros)
    cf = _conv(p['v2c2'], vf, cf, dst, src, dst2, src2, edge_b, zeros)

    con_out = _head(cf, p['con_W1'], p['con_b1'], p['con_W2'])
    var_out = _head(vf, p['varm_W1'], p['varm_b1'], p['varm_W2'])
    return jnp.squeeze(con_out, -1), jnp.squeeze(var_out, -1)


# speedup vs baseline: 1.5813x; 1.0978x over previous
import functools

import jax
import jax.numpy as jnp
from jax import lax
from jax.experimental import pallas as pl
from jax.experimental.pallas import tpu as pltpu
from jax.experimental.pallas import tpu_sc as plsc

F32 = jnp.float32
EMBD = 64
PK = 2 * EMBD
NBLK = 1000
EBLK = 2000
EPS = 1e-5

SC_CORES = 2
SC_SUBCORES = 16
NWORK = SC_CORES * SC_SUBCORES
GCHUNK = 128


def _mm(x, w):
    return lax.dot_general(x, w, (((1,), (1,)), ((), ())),
                           preferred_element_type=F32)


def _lnf(x, g, b):
    m = jnp.mean(x, axis=-1, keepdims=True)
    v = jnp.mean((x - m) ** 2, axis=-1, keepdims=True)
    return (x - m) / jnp.sqrt(v + EPS) * g + b


def _bspec(shape, imap):
    return pl.BlockSpec(shape, imap)


def _embed_body(xa_ref, xb_ref, g_ref, b_ref, w1_ref, b1_ref, w2_ref, b2_ref,
                out_ref, cs_ref):
    def half(x):
        h = _lnf(x, g_ref[...], b_ref[...])
        h = jnp.maximum(_mm(h, w1_ref[...]) + b1_ref[...], 0.0)
        return jnp.maximum(_mm(h, w2_ref[...]) + b2_ref[...], 0.0)

    ha = half(xa_ref[...])
    hb = half(xb_ref[...])
    out_ref[:, :EMBD] = ha
    out_ref[:, EMBD:] = hb

    @pl.when(pl.program_id(0) == 0)
    def _():
        cs_ref[...] = jnp.zeros_like(cs_ref)

    cs_ref[:, :EMBD] += jnp.sum(ha, axis=0, keepdims=True)
    cs_ref[:, EMBD:] += jnp.sum(hb, axis=0, keepdims=True)


def _embed(x, g, b, w1, b1, w2, b2):
    n, fin = x.shape
    n2 = n // 2
    grid = n2 // NBLK
    out, cs = pl.pallas_call(
        _embed_body,
        grid=(grid,),
        in_specs=[
            _bspec((NBLK, fin), lambda i: (i, 0)),
            _bspec((NBLK, fin), lambda i, g=grid: (i + g, 0)),
            _bspec((1, fin), lambda i: (0, 0)),
            _bspec((1, fin), lambda i: (0, 0)),
            _bspec((EMBD, fin), lambda i: (0, 0)),
            _bspec((1, EMBD), lambda i: (0, 0)),
            _bspec((EMBD, EMBD), lambda i: (0, 0)),
            _bspec((1, EMBD), lambda i: (0, 0)),
        ],
        out_specs=[
            _bspec((NBLK, PK), lambda i: (i, 0)),
            _bspec((1, PK), lambda i: (0, 0)),
        ],
        out_shape=[
            jax.ShapeDtypeStruct((n2, PK), F32),
            jax.ShapeDtypeStruct((1, PK), F32),
        ],
        compiler_params=pltpu.CompilerParams(
            dimension_semantics=("arbitrary",)),
    )(x, x, g.reshape(1, fin), b.reshape(1, fin), w1, b1.reshape(1, EMBD),
      w2, b2.reshape(1, EMBD))
    return out, cs


def _se_body(x_ref, cs_ref, w1_ref, w2_ref, out_ref, *, inv_n):
    mean = (cs_ref[:, :EMBD] + cs_ref[:, EMBD:]) * inv_n
    h = jnp.maximum(_mm(mean, w1_ref[...]), 0.0)
    w = jax.nn.sigmoid(_mm(h, w2_ref[...]))
    out_ref[:, :EMBD] = x_ref[:, :EMBD] * w
    out_ref[:, EMBD:] = x_ref[:, EMBD:] * w


def _se_apply(x, cs, w1, w2):
    n2 = x.shape[0]
    sq = w1.shape[0]
    return pl.pallas_call(
        functools.partial(_se_body, inv_n=0.5 / n2),
        grid=(n2 // NBLK,),
        in_specs=[
            _bspec((NBLK, PK), lambda i: (i, 0)),
            _bspec((1, PK), lambda i: (0, 0)),
            _bspec((sq, EMBD), lambda i: (0, 0)),
            _bspec((EMBD, sq), lambda i: (0, 0)),
        ],
        out_specs=_bspec((NBLK, PK), lambda i: (i, 0)),
        out_shape=jax.ShapeDtypeStruct((n2, PK), F32),
    )(x, cs, w1, w2)


def _pq_body(r_ref, l_ref, wl_ref, blc_ref, wr_ref, p_ref, q_ref):
    p_ref[:, :EMBD] = _mm(r_ref[:, :EMBD], wl_ref[...]) + blc_ref[...]
    p_ref[:, EMBD:] = _mm(r_ref[:, EMBD:], wl_ref[...]) + blc_ref[...]
    q_ref[:, :EMBD] = _mm(l_ref[:, :EMBD], wr_ref[...])
    q_ref[:, EMBD:] = _mm(l_ref[:, EMBD:], wr_ref[...])


def _pq(right, left, wl, blc, wr):
    n2 = right.shape[0]
    return pl.pallas_call(
        _pq_body,
        grid=(n2 // NBLK,),
        in_specs=[
            _bspec((NBLK, PK), lambda i: (i, 0)),
            _bspec((NBLK, PK), lambda i: (i, 0)),
            _bspec((EMBD, EMBD), lambda i: (0, 0)),
            _bspec((1, EMBD), lambda i: (0, 0)),
            _bspec((EMBD, EMBD), lambda i: (0, 0)),
        ],
        out_specs=[
            _bspec((NBLK, PK), lambda i: (i, 0)),
            _bspec((NBLK, PK), lambda i: (i, 0)),
        ],
        out_shape=[
            jax.ShapeDtypeStruct((n2, PK), F32),
            jax.ShapeDtypeStruct((n2, PK), F32),
        ],
    )(right, left, wl, blc, wr)


def _edge_body(pg_ref, qg_ref, d_ref, s_ref, g_ref, b_ref, wf_ref, bf_ref,
               msg_ref, *, n_half):
    pd = d_ref[...] >= n_half
    ps = s_ref[...] >= n_half
    p = jnp.where(pd, pg_ref[:, EMBD:], pg_ref[:, :EMBD])
    q = jnp.where(ps, qg_ref[:, EMBD:], qg_ref[:, :EMBD])
    h = _lnf(p + q, g_ref[...], b_ref[...])
    h = jnp.maximum(h, 0.0)
    m = _mm(h, wf_ref[...]) + bf_ref[...]
    msg_ref[:, :EMBD] = jnp.where(pd, 0.0, m)
    msg_ref[:, EMBD:] = jnp.where(pd, m, 0.0)


def _edge_mlp(pg, qg, d2, s2, g, b, wf, bf, n_half):
    e = pg.shape[0]
    return pl.pallas_call(
        functools.partial(_edge_body, n_half=n_half),
        grid=(e // EBLK,),
        in_specs=[
            _bspec((EBLK, PK), lambda i: (i, 0)),
            _bspec((EBLK, PK), lambda i: (i, 0)),
            _bspec((EBLK, 1), lambda i: (i, 0)),
            _bspec((EBLK, 1), lambda i: (i, 0)),
            _bspec((1, EMBD), lambda i: (0, 0)),
            _bspec((1, EMBD), lambda i: (0, 0)),
            _bspec((EMBD, EMBD), lambda i: (0, 0)),
            _bspec((1, EMBD), lambda i: (0, 0)),
        ],
        out_specs=_bspec((EBLK, PK), lambda i: (i, 0)),
        out_shape=jax.ShapeDtypeStruct((e, PK), F32),
    )(pg, qg, d2, s2, g.reshape(1, EMBD), b.reshape(1, EMBD), wf,
      bf.reshape(1, EMBD))


def _post_body(agg_ref, r_ref, g_ref, b_ref, w1a_ref, w1b_ref, bo1_ref,
               w2_ref, bo2_ref, out_ref):
    def half(sl):
        post = _lnf(agg_ref[:, sl], g_ref[...], b_ref[...])
        h = (_mm(post, w1a_ref[...]) + _mm(r_ref[:, sl], w1b_ref[...])
             + bo1_ref[...])
        h = jnp.maximum(h, 0.0)
        return _mm(h, w2_ref[...]) + bo2_ref[...]

    out_ref[:, :EMBD] = half(slice(0, EMBD))
    out_ref[:, EMBD:] = half(slice(EMBD, PK))


def _post(agg, right, g, b, w1a, w1b, bo1, w2, bo2):
    n2 = agg.shape[0]
    return pl.pallas_call(
        _post_body,
        grid=(n2 // NBLK,),
        in_specs=[
            _bspec((NBLK, PK), lambda i: (i, 0)),
            _bspec((NBLK, PK), lambda i: (i, 0)),
            _bspec((1, EMBD), lambda i: (0, 0)),
            _bspec((1, EMBD), lambda i: (0, 0)),
            _bspec((EMBD, EMBD), lambda i: (0, 0)),
            _bspec((EMBD, EMBD), lambda i: (0, 0)),
            _bspec((1, EMBD), lambda i: (0, 0)),
            _bspec((EMBD, EMBD), lambda i: (0, 0)),
            _bspec((1, EMBD), lambda i: (0, 0)),
        ],
        out_specs=_bspec((NBLK, PK), lambda i: (i, 0)),
        out_shape=jax.ShapeDtypeStruct((n2, PK), F32),
    )(agg, right, g.reshape(1, EMBD), b.reshape(1, EMBD), w1a, w1b,
      bo1.reshape(1, EMBD), w2, bo2.reshape(1, EMBD))


def _head_body(x_ref, w1_ref, b1_ref, w2_ref, oa_ref, ob_ref):
    def half(sl):
        h = jnp.maximum(_mm(x_ref[:, sl], w1_ref[...]) + b1_ref[...], 0.0)
        return jax.nn.sigmoid(_mm(h, w2_ref[...]) * (1.0 / 0.6))

    oa_ref[...] = half(slice(0, EMBD))
    ob_ref[...] = half(slice(EMBD, PK))


def _head(x, w1, b1, w2):
    n2 = x.shape[0]
    oa, ob = pl.pallas_call(
        _head_body,
        grid=(n2 // NBLK,),
        in_specs=[
            _bspec((NBLK, PK), lambda i: (i, 0)),
            _bspec((EMBD, EMBD), lambda i: (0, 0)),
            _bspec((1, EMBD), lambda i: (0, 0)),
            _bspec((1, EMBD), lambda i: (0, 0)),
        ],
        out_specs=[
            _bspec((NBLK, 1), lambda i: (i, 0)),
            _bspec((NBLK, 1), lambda i: (i, 0)),
        ],
        out_shape=[
            jax.ShapeDtypeStruct((n2, 1), F32),
            jax.ShapeDtypeStruct((n2, 1), F32),
        ],
    )(x, w1, b1.reshape(1, EMBD), w2)
    return jnp.concatenate([oa, ob], axis=0)


def _sc_gather(p_tab, q_tab, d_idx, s_idx):
    e = d_idx.shape[0]
    n_half = p_tab.shape[0]
    per_w = e // NWORK
    iters = per_w // GCHUNK
    tail = per_w - iters * GCHUNK
    mesh = plsc.VectorSubcoreMesh(core_axis_name="c", subcore_axis_name="s")

    @functools.partial(
        pl.kernel, mesh=mesh,
        out_type=(jax.ShapeDtypeStruct((e, PK), F32),
                  jax.ShapeDtypeStruct((e, PK), F32)),
        scratch_types=[
            pltpu.VMEM((GCHUNK,), jnp.int32),
            pltpu.VMEM((GCHUNK,), jnp.int32),
            pltpu.VMEM((GCHUNK, PK), F32),
            pltpu.VMEM((GCHUNK, PK), F32),
            pltpu.VMEM((tail,), jnp.int32),
            pltpu.VMEM((tail,), jnp.int32),
            pltpu.VMEM((tail, PK), F32),
            pltpu.VMEM((tail, PK), F32),
            pltpu.SemaphoreType.DMA,
            pltpu.SemaphoreType.DMA,
        ],
    )
    def k(p_hbm, q_hbm, d_hbm, s_hbm, pg_hbm, qg_hbm,
          di, si, pr, qr, dit, sit, prt, qrt, sem1, sem2):
        wid = lax.axis_index("s") * SC_CORES + lax.axis_index("c")
        base = wid * per_w

        def physify(ref, count):
            for kk in range(count // 16):
                v = ref[pl.ds(kk * 16, 16)]
                ref[pl.ds(kk * 16, 16)] = jnp.where(v >= n_half, v - n_half, v)

        def step(off, dref, sref, prref, qrref, count):
            pltpu.sync_copy(d_hbm.at[pl.ds(off, count)], dref)
            pltpu.sync_copy(s_hbm.at[pl.ds(off, count)], sref)
            physify(dref, count)
            physify(sref, count)
            c1 = pltpu.async_copy(p_hbm.at[dref], prref, sem1)
            c2 = pltpu.async_copy(q_hbm.at[sref], qrref, sem2)
            c1.wait()
            c2.wait()
            pltpu.sync_copy(prref, pg_hbm.at[pl.ds(off, count)])
            pltpu.sync_copy(qrref, qg_hbm.at[pl.ds(off, count)])

        def body(j, carry):
            step(base + j * GCHUNK, di, si, pr, qr, GCHUNK)
            return carry

        lax.fori_loop(0, iters, body, 0)
        if tail:
            step(base + iters * GCHUNK, dit, sit, prt, qrt, tail)

    return k(p_tab, q_tab, d_idx, s_idx)


def _sc_scatter(msg, d_idx, zeros, n_half):
    e = d_idx.shape[0]
    half = n_half // SC_CORES
    pad = zeros.shape[0]
    rows_pt = pad // SC_SUBCORES
    chunk = 112
    per_t = e // SC_SUBCORES
    iters = per_t // chunk
    tail = per_t - iters * chunk
    depth = 2
    supers = iters // depth
    rem = iters - supers * depth
    mesh = plsc.VectorSubcoreMesh(core_axis_name="c", subcore_axis_name="s")

    @functools.partial(
        pl.kernel, mesh=mesh,
        out_type=jax.ShapeDtypeStruct((SC_CORES, pad, PK), F32),
        scratch_types=(
            [pltpu.VMEM((chunk,), jnp.int32)] * depth
            + [pltpu.VMEM((chunk, PK), F32)] * depth
            + [pltpu.SemaphoreType.DMA] * (2 * depth)
            + [pltpu.VMEM((max(tail, 16),), jnp.int32),
               pltpu.VMEM_SHARED((pad, PK), F32)]
        ),
    )
    def k(msg_hbm, d_hbm, z_hbm, out_hbm, *refs):
        draws = refs[:depth]
        mbufs = refs[depth:2 * depth]
        lsems = refs[2 * depth:3 * depth]
        ssems = refs[3 * depth:4 * depth]
        drawt, acc = refs[4 * depth:]
        cid = lax.axis_index("c")
        sid = lax.axis_index("s")
        base = cid * half
        dump = half + sid

        zr = sid * rows_pt
        pltpu.sync_copy(z_hbm.at[pl.ds(zr, rows_pt)],
                        acc.at[pl.ds(zr, rows_pt)])
        plsc.subcore_barrier()

        def localize(ref, count):
            for kk in range(count // 16):
                v = ref[pl.ds(kk * 16, 16)]
                v = jnp.where(v >= n_half, v - n_half, v)
                l = v - base
                ok = (l >= 0) & (l < half)
                ref[pl.ds(kk * 16, 16)] = jnp.where(ok, l, dump)

        def pipe_group(off0):
            loads = []
            for b in range(depth):
                off = off0 + b * chunk
                c1 = pltpu.async_copy(d_hbm.at[pl.ds(off, chunk)],
                                      draws[b], lsems[b])
                c2 = pltpu.async_copy(msg_hbm.at[pl.ds(off, chunk)],
                                      mbufs[b], lsems[b])
                loads.append((c1, c2))
            scats = []
            for b in range(depth):
                loads[b][0].wait()
                loads[b][1].wait()
                localize(draws[b], chunk)
                scats.append(pltpu.async_copy(mbufs[b], acc.at[draws[b]],
                                              ssems[b], add=True))
            for s in scats:
                s.wait()

        def body(j, carry):
            pipe_group(sid * per_t + j * (depth * chunk))
            return carry

        lax.fori_loop(0, supers, body, 0)
        for r in range(rem):
            off = sid * per_t + (supers * depth + r) * chunk
            pltpu.sync_copy(d_hbm.at[pl.ds(off, chunk)], draws[0])
            pltpu.sync_copy(msg_hbm.at[pl.ds(off, chunk)], mbufs[0])
            localize(draws[0], chunk)
            pltpu.sync_copy(mbufs[0], acc.at[draws[0]], add=True)
        if tail:
            off = sid * per_t + iters * chunk
            pltpu.sync_copy(d_hbm.at[pl.ds(off, tail)], drawt)
            pltpu.sync_copy(msg_hbm.at[pl.ds(off, tail)],
                            mbufs[0].at[pl.ds(0, tail)])
            localize(drawt, tail)
            pltpu.sync_copy(mbufs[0].at[pl.ds(0, tail)],
                            acc.at[drawt], add=True)

        plsc.subcore_barrier()
        pltpu.sync_copy(acc.at[pl.ds(zr, rows_pt)],
                        out_hbm.at[cid, pl.ds(zr, rows_pt)])

    out = k(msg, d_idx, zeros)
    return jnp.concatenate([out[0, :half], out[1, :half]], axis=0)


def _conv(cp, left, right, s_idx, d_idx, s2, d2, edge_b, zeros):
    n_half = right.shape[0]
    blc = (cp['bl'] + edge_b * cp['We'][:, 0]).reshape(1, EMBD)
    p_tab, q_tab = _pq(right, left, cp['Wl'], blc, cp['Wr'])
    pg, qg = _sc_gather(p_tab, q_tab, d_idx, s_idx)
    msg = _edge_mlp(pg, qg, d2, s2, cp['ln_g'], cp['ln_b'], cp['Wf'],
                    cp['bf'], n_half)
    agg = _sc_scatter(msg, d_idx, zeros, n_half)
    return _post(agg, right, cp['pc_g'], cp['pc_b'],
                 cp['Wo1'][:, :EMBD], cp['Wo1'][:, EMBD:], cp['bo1'],
                 cp['Wo2'], cp['bo2'])


def kernel(params, constraint_features, edge_indices, edge_features,
           variable_features):
    p = params
    src = edge_indices[0]
    dst = edge_indices[1]
    e = src.shape[0]
    src2 = src.reshape(e, 1)
    dst2 = dst.reshape(e, 1)
    n_c = constraint_features.shape[0]
    edge_b = p['edge_ln_b'][0]

    half = n_c // 2 // SC_CORES
    pad = ((half + SC_SUBCORES) + 127) // 128 * 128
    zeros = jnp.zeros((pad, PK), F32)

    cf, cs = _embed(constraint_features, p['cons_ln_g'], p['cons_ln_b'],
                    p['cons_W1'], p['cons_b1'], p['cons_W2'], p['cons_b2'])
    cf = _se_apply(cf, cs, p['se_con_W1'], p['se_con_W2'])
    vf, vs = _embed(variable_features, p['var_ln_g'], p['var_ln_b'],
                    p['var_W1'], p['var_b1'], p['var_W2'], p['var_b2'])
    vf = _se_apply(vf, vs, p['se_var_W1'], p['se_var_W2'])

    vf = _conv(p['c2v'], cf, vf, src, dst, src2, dst2, edge_b, zeros)
    cf = _conv(p['v2c'], vf, cf, dst, src, dst2, src2, edge_b, zeros)
    vf = _conv(p['c2v2'], cf, vf, src, dst, src2, dst2, edge_b, zeros)
    cf = _conv(p['v2c2'], vf, cf, dst, src, dst2, src2, edge_b, zeros)

    con_out = _head(cf, p['con_W1'], p['con_b1'], p['con_W2'])
    var_out = _head(vf, p['varm_W1'], p['varm_b1'], p['varm_W2'])
    return jnp.squeeze(con_out, -1), jnp.squeeze(var_out, -1)


# speedup vs baseline: 1.9786x; 1.2512x over previous
import functools

import jax
import jax.numpy as jnp
from jax import lax
from jax.experimental import pallas as pl
from jax.experimental.pallas import tpu as pltpu
from jax.experimental.pallas import tpu_sc as plsc

F32 = jnp.float32
EMBD = 64
PK = 2 * EMBD
NBLK = 1000
EBLK = 2000
EPS = 1e-5

SC_CORES = 2
SC_SUBCORES = 16
NWORK = SC_CORES * SC_SUBCORES
GCHUNK = 128


def _mm(x, w):
    return lax.dot_general(x, w, (((1,), (1,)), ((), ())),
                           preferred_element_type=F32)


def _lnf(x, g, b):
    m = jnp.mean(x, axis=-1, keepdims=True)
    v = jnp.mean((x - m) ** 2, axis=-1, keepdims=True)
    return (x - m) / jnp.sqrt(v + EPS) * g + b


def _bspec(shape, imap):
    return pl.BlockSpec(shape, imap)


def _embed_body(xa_ref, xb_ref, g_ref, b_ref, w1_ref, b1_ref, w2_ref, b2_ref,
                out_ref, cs_ref):
    def half(x):
        h = _lnf(x, g_ref[...], b_ref[...])
        h = jnp.maximum(_mm(h, w1_ref[...]) + b1_ref[...], 0.0)
        return jnp.maximum(_mm(h, w2_ref[...]) + b2_ref[...], 0.0)

    ha = half(xa_ref[...])
    hb = half(xb_ref[...])
    out_ref[:, :EMBD] = ha
    out_ref[:, EMBD:] = hb

    @pl.when(pl.program_id(0) == 0)
    def _():
        cs_ref[...] = jnp.zeros_like(cs_ref)

    cs_ref[:, :EMBD] += jnp.sum(ha, axis=0, keepdims=True)
    cs_ref[:, EMBD:] += jnp.sum(hb, axis=0, keepdims=True)


def _embed(x, g, b, w1, b1, w2, b2):
    n, fin = x.shape
    n2 = n // 2
    grid = n2 // NBLK
    out, cs = pl.pallas_call(
        _embed_body,
        grid=(grid,),
        in_specs=[
            _bspec((NBLK, fin), lambda i: (i, 0)),
            _bspec((NBLK, fin), lambda i, g=grid: (i + g, 0)),
            _bspec((1, fin), lambda i: (0, 0)),
            _bspec((1, fin), lambda i: (0, 0)),
            _bspec((EMBD, fin), lambda i: (0, 0)),
            _bspec((1, EMBD), lambda i: (0, 0)),
            _bspec((EMBD, EMBD), lambda i: (0, 0)),
            _bspec((1, EMBD), lambda i: (0, 0)),
        ],
        out_specs=[
            _bspec((NBLK, PK), lambda i: (i, 0)),
            _bspec((1, PK), lambda i: (0, 0)),
        ],
        out_shape=[
            jax.ShapeDtypeStruct((n2, PK), F32),
            jax.ShapeDtypeStruct((1, PK), F32),
        ],
        compiler_params=pltpu.CompilerParams(
            dimension_semantics=("arbitrary",)),
    )(x, x, g.reshape(1, fin), b.reshape(1, fin), w1, b1.reshape(1, EMBD),
      w2, b2.reshape(1, EMBD))
    return out, cs


def _se_body(x_ref, cs_ref, w1_ref, w2_ref, out_ref, *, inv_n):
    mean = (cs_ref[:, :EMBD] + cs_ref[:, EMBD:]) * inv_n
    h = jnp.maximum(_mm(mean, w1_ref[...]), 0.0)
    w = jax.nn.sigmoid(_mm(h, w2_ref[...]))
    out_ref[:, :EMBD] = x_ref[:, :EMBD] * w
    out_ref[:, EMBD:] = x_ref[:, EMBD:] * w


def _se_apply(x, cs, w1, w2):
    n2 = x.shape[0]
    sq = w1.shape[0]
    return pl.pallas_call(
        functools.partial(_se_body, inv_n=0.5 / n2),
        grid=(n2 // NBLK,),
        in_specs=[
            _bspec((NBLK, PK), lambda i: (i, 0)),
            _bspec((1, PK), lambda i: (0, 0)),
            _bspec((sq, EMBD), lambda i: (0, 0)),
            _bspec((EMBD, sq), lambda i: (0, 0)),
        ],
        out_specs=_bspec((NBLK, PK), lambda i: (i, 0)),
        out_shape=jax.ShapeDtypeStruct((n2, PK), F32),
    )(x, cs, w1, w2)


def _pq_body(r_ref, l_ref, wl_ref, blc_ref, wr_ref, p_ref, q_ref):
    p_ref[:, :EMBD] = _mm(r_ref[:, :EMBD], wl_ref[...]) + blc_ref[...]
    p_ref[:, EMBD:] = _mm(r_ref[:, EMBD:], wl_ref[...]) + blc_ref[...]
    q_ref[:, :EMBD] = _mm(l_ref[:, :EMBD], wr_ref[...])
    q_ref[:, EMBD:] = _mm(l_ref[:, EMBD:], wr_ref[...])


def _pq(right, left, wl, blc, wr):
    n2 = right.shape[0]
    return pl.pallas_call(
        _pq_body,
        grid=(n2 // NBLK,),
        in_specs=[
            _bspec((NBLK, PK), lambda i: (i, 0)),
            _bspec((NBLK, PK), lambda i: (i, 0)),
            _bspec((EMBD, EMBD), lambda i: (0, 0)),
            _bspec((1, EMBD), lambda i: (0, 0)),
            _bspec((EMBD, EMBD), lambda i: (0, 0)),
        ],
        out_specs=[
            _bspec((NBLK, PK), lambda i: (i, 0)),
            _bspec((NBLK, PK), lambda i: (i, 0)),
        ],
        out_shape=[
            jax.ShapeDtypeStruct((n2, PK), F32),
            jax.ShapeDtypeStruct((n2, PK), F32),
        ],
    )(right, left, wl, blc, wr)


def _edge_body(pg_ref, qg_ref, d_ref, s_ref, g_ref, b_ref, wf_ref, bf_ref,
               msg_ref, *, n_half):
    pd = d_ref[...] >= n_half
    ps = s_ref[...] >= n_half
    p = jnp.where(pd, pg_ref[:, EMBD:], pg_ref[:, :EMBD])
    q = jnp.where(ps, qg_ref[:, EMBD:], qg_ref[:, :EMBD])
    h = _lnf(p + q, g_ref[...], b_ref[...])
    h = jnp.maximum(h, 0.0)
    m = _mm(h, wf_ref[...]) + bf_ref[...]
    msg_ref[:, :EMBD] = jnp.where(pd, 0.0, m)
    msg_ref[:, EMBD:] = jnp.where(pd, m, 0.0)


def _edge_mlp(pg, qg, d2, s2, g, b, wf, bf, n_half):
    e = pg.shape[0]
    return pl.pallas_call(
        functools.partial(_edge_body, n_half=n_half),
        grid=(e // EBLK,),
        in_specs=[
            _bspec((EBLK, PK), lambda i: (i, 0)),
            _bspec((EBLK, PK), lambda i: (i, 0)),
            _bspec((EBLK, 1), lambda i: (i, 0)),
            _bspec((EBLK, 1), lambda i: (i, 0)),
            _bspec((1, EMBD), lambda i: (0, 0)),
            _bspec((1, EMBD), lambda i: (0, 0)),
            _bspec((EMBD, EMBD), lambda i: (0, 0)),
            _bspec((1, EMBD), lambda i: (0, 0)),
        ],
        out_specs=_bspec((EBLK, PK), lambda i: (i, 0)),
        out_shape=jax.ShapeDtypeStruct((e, PK), F32),
    )(pg, qg, d2, s2, g.reshape(1, EMBD), b.reshape(1, EMBD), wf,
      bf.reshape(1, EMBD))


def _post_body(agga_ref, aggb_ref, r_ref, g_ref, b_ref, w1a_ref, w1b_ref,
               bo1_ref, w2_ref, bo2_ref, out_ref):
    def half(sl):
        post = _lnf(agga_ref[:, sl] + aggb_ref[:, sl], g_ref[...], b_ref[...])
        h = (_mm(post, w1a_ref[...]) + _mm(r_ref[:, sl], w1b_ref[...])
             + bo1_ref[...])
        h = jnp.maximum(h, 0.0)
        return _mm(h, w2_ref[...]) + bo2_ref[...]

    out_ref[:, :EMBD] = half(slice(0, EMBD))
    out_ref[:, EMBD:] = half(slice(EMBD, PK))


def _post(agga, aggb, right, g, b, w1a, w1b, bo1, w2, bo2):
    n2 = agga.shape[0]
    return pl.pallas_call(
        _post_body,
        grid=(n2 // NBLK,),
        in_specs=[
            _bspec((NBLK, PK), lambda i: (i, 0)),
            _bspec((NBLK, PK), lambda i: (i, 0)),
            _bspec((NBLK, PK), lambda i: (i, 0)),
            _bspec((1, EMBD), lambda i: (0, 0)),
            _bspec((1, EMBD), lambda i: (0, 0)),
            _bspec((EMBD, EMBD), lambda i: (0, 0)),
            _bspec((EMBD, EMBD), lambda i: (0, 0)),
            _bspec((1, EMBD), lambda i: (0, 0)),
            _bspec((EMBD, EMBD), lambda i: (0, 0)),
            _bspec((1, EMBD), lambda i: (0, 0)),
        ],
        out_specs=_bspec((NBLK, PK), lambda i: (i, 0)),
        out_shape=jax.ShapeDtypeStruct((n2, PK), F32),
    )(agga, aggb, right, g.reshape(1, EMBD), b.reshape(1, EMBD), w1a, w1b,
      bo1.reshape(1, EMBD), w2, bo2.reshape(1, EMBD))


def _head_body(x_ref, w1_ref, b1_ref, w2_ref, oa_ref, ob_ref):
    def half(sl):
        h = jnp.maximum(_mm(x_ref[:, sl], w1_ref[...]) + b1_ref[...], 0.0)
        return jax.nn.sigmoid(_mm(h, w2_ref[...]) * (1.0 / 0.6))

    oa_ref[...] = half(slice(0, EMBD))
    ob_ref[...] = half(slice(EMBD, PK))


def _head(x, w1, b1, w2):
    n2 = x.shape[0]
    oa, ob = pl.pallas_call(
        _head_body,
        grid=(n2 // NBLK,),
        in_specs=[
            _bspec((NBLK, PK), lambda i: (i, 0)),
            _bspec((EMBD, EMBD), lambda i: (0, 0)),
            _bspec((1, EMBD), lambda i: (0, 0)),
            _bspec((1, EMBD), lambda i: (0, 0)),
        ],
        out_specs=[
            _bspec((NBLK, 1), lambda i: (i, 0)),
            _bspec((NBLK, 1), lambda i: (i, 0)),
        ],
        out_shape=[
            jax.ShapeDtypeStruct((n2, 1), F32),
            jax.ShapeDtypeStruct((n2, 1), F32),
        ],
    )(x, w1, b1.reshape(1, EMBD), w2)
    return jnp.concatenate([oa, ob], axis=0)


def _sc_gather(p_tab, q_tab, d_idx, s_idx):
    e = d_idx.shape[0]
    n_half = p_tab.shape[0]
    per_w = e // NWORK
    iters = per_w // GCHUNK
    tail = per_w - iters * GCHUNK
    tail_p = (tail + 15) // 16 * 16
    mesh = plsc.VectorSubcoreMesh(core_axis_name="c", subcore_axis_name="s")

    @functools.partial(
        pl.kernel, mesh=mesh,
        out_type=(jax.ShapeDtypeStruct((e, PK), F32),
                  jax.ShapeDtypeStruct((e, PK), F32)),
        scratch_types=[
            pltpu.VMEM((GCHUNK,), jnp.int32),
            pltpu.VMEM((GCHUNK,), jnp.int32),
            pltpu.VMEM((GCHUNK, PK), F32),
            pltpu.VMEM((GCHUNK, PK), F32),
            pltpu.VMEM((max(tail_p, 16),), jnp.int32),
            pltpu.VMEM((max(tail_p, 16),), jnp.int32),
            pltpu.VMEM((max(tail_p, 16), PK), F32),
            pltpu.VMEM((max(tail_p, 16), PK), F32),
            pltpu.SemaphoreType.DMA,
            pltpu.SemaphoreType.DMA,
        ],
    )
    def k(p_hbm, q_hbm, d_hbm, s_hbm, pg_hbm, qg_hbm,
          di, si, pr, qr, dit, sit, prt, qrt, sem1, sem2):
        wid = lax.axis_index("s") * SC_CORES + lax.axis_index("c")
        base = wid * per_w

        def physify(ref, count):
            for kk in range((count + 15) // 16):
                v = ref[pl.ds(kk * 16, 16)]
                v = jnp.where(v >= n_half, v - n_half, v)
                if (kk + 1) * 16 > count:
                    lane = lax.iota(jnp.int32, 16)
                    v = jnp.where(lane < count - kk * 16, v, 0)
                ref[pl.ds(kk * 16, 16)] = v

        def step(off, dref, sref, prref, qrref, count):
            pltpu.sync_copy(d_hbm.at[pl.ds(off, count)],
                            dref.at[pl.ds(0, count)])
            pltpu.sync_copy(s_hbm.at[pl.ds(off, count)],
                            sref.at[pl.ds(0, count)])
            physify(dref, count)
            physify(sref, count)
            c1 = pltpu.async_copy(p_hbm.at[dref], prref, sem1)
            c2 = pltpu.async_copy(q_hbm.at[sref], qrref, sem2)
            c1.wait()
            c2.wait()
            pltpu.sync_copy(prref.at[pl.ds(0, count)],
                            pg_hbm.at[pl.ds(off, count)])
            pltpu.sync_copy(qrref.at[pl.ds(0, count)],
                            qg_hbm.at[pl.ds(off, count)])

        def body(j, carry):
            step(base + j * GCHUNK, di, si, pr, qr, GCHUNK)
            return carry

        lax.fori_loop(0, iters, body, 0)
        if tail:
            step(base + iters * GCHUNK, dit, sit, prt, qrt, tail)

    return k(p_tab, q_tab, d_idx, s_idx)


def _sc_scatter(msg, d_idx, zeros, n_half):
    e = d_idx.shape[0]
    half = n_half // SC_CORES
    pad = zeros.shape[0]
    rows_pt = pad // SC_SUBCORES
    chunk = 112
    per_t = e // SC_SUBCORES
    iters = per_t // chunk
    tail = per_t - iters * chunk
    depth = 2
    supers = iters // depth
    rem = iters - supers * depth
    mesh = plsc.VectorSubcoreMesh(core_axis_name="c", subcore_axis_name="s")

    @functools.partial(
        pl.kernel, mesh=mesh,
        out_type=jax.ShapeDtypeStruct((SC_CORES, pad, PK), F32),
        scratch_types=(
            [pltpu.VMEM((chunk,), jnp.int32)] * depth
            + [pltpu.VMEM((chunk, PK), F32)] * depth
            + [pltpu.SemaphoreType.DMA] * (2 * depth)
            + [pltpu.VMEM((max((tail + 15) // 16 * 16, 16),), jnp.int32),
               pltpu.VMEM_SHARED((pad, PK), F32)]
        ),
    )
    def k(msg_hbm, d_hbm, z_hbm, out_hbm, *refs):
        draws = refs[:depth]
        mbufs = refs[depth:2 * depth]
        lsems = refs[2 * depth:3 * depth]
        ssems = refs[3 * depth:4 * depth]
        drawt, acc = refs[4 * depth:]
        cid = lax.axis_index("c")
        sid = lax.axis_index("s")
        base = cid * half
        dump = half + sid

        zr = sid * rows_pt
        pltpu.sync_copy(z_hbm.at[pl.ds(zr, rows_pt)],
                        acc.at[pl.ds(zr, rows_pt)])
        plsc.subcore_barrier()

        def localize(ref, count):
            for kk in range((count + 15) // 16):
                v = ref[pl.ds(kk * 16, 16)]
                v = jnp.where(v >= n_half, v - n_half, v)
                l = v - base
                ok = (l >= 0) & (l < half)
                if (kk + 1) * 16 > count:
                    lane = lax.iota(jnp.int32, 16)
                    ok = ok & (lane < count - kk * 16)
                ref[pl.ds(kk * 16, 16)] = jnp.where(ok, l, dump)

        def pipe_group(off0):
            loads = []
            for b in range(depth):
                off = off0 + b * chunk
                c1 = pltpu.async_copy(d_hbm.at[pl.ds(off, chunk)],
                                      draws[b], lsems[b])
                c2 = pltpu.async_copy(msg_hbm.at[pl.ds(off, chunk)],
                                      mbufs[b], lsems[b])
                loads.append((c1, c2))
            scats = []
            for b in range(depth):
                loads[b][0].wait()
                loads[b][1].wait()
                localize(draws[b], chunk)
                scats.append(pltpu.async_copy(mbufs[b], acc.at[draws[b]],
                                              ssems[b], add=True))
            for s in scats:
                s.wait()

        def body(j, carry):
            pipe_group(sid * per_t + j * (depth * chunk))
            return carry

        lax.fori_loop(0, supers, body, 0)
        for r in range(rem):
            off = sid * per_t + (supers * depth + r) * chunk
            pltpu.sync_copy(d_hbm.at[pl.ds(off, chunk)], draws[0])
            pltpu.sync_copy(msg_hbm.at[pl.ds(off, chunk)], mbufs[0])
            localize(draws[0], chunk)
            pltpu.sync_copy(mbufs[0], acc.at[draws[0]], add=True)
        if tail:
            tail_p = (tail + 15) // 16 * 16
            off = sid * per_t + iters * chunk
            pltpu.sync_copy(d_hbm.at[pl.ds(off, tail)],
                            drawt.at[pl.ds(0, tail)])
            pltpu.sync_copy(msg_hbm.at[pl.ds(off, tail)],
                            mbufs[0].at[pl.ds(0, tail)])
            localize(drawt, tail)
            pltpu.sync_copy(mbufs[0].at[pl.ds(0, tail_p)],
                            acc.at[drawt], add=True)

        plsc.subcore_barrier()
        pltpu.sync_copy(acc.at[pl.ds(zr, rows_pt)],
                        out_hbm.at[cid, pl.ds(zr, rows_pt)])

    out = k(msg, d_idx, zeros)
    return jnp.concatenate([out[0, :half], out[1, :half]], axis=0)


def _conv(cp, left, right, s_idx, d_idx, s2, d2, edge_b, zeros):
    n_half = right.shape[0]
    e = d_idx.shape[0]
    h = ((e // 2) // 32000 * 32000) or (e // 2)
    blc = (cp['bl'] + edge_b * cp['We'][:, 0]).reshape(1, EMBD)
    p_tab, q_tab = _pq(right, left, cp['Wl'], blc, cp['Wr'])
    pg1, qg1 = _sc_gather(p_tab, q_tab, d_idx[:h], s_idx[:h])
    msg1 = _edge_mlp(pg1, qg1, d2[:h], s2[:h], cp['ln_g'], cp['ln_b'],
                     cp['Wf'], cp['bf'], n_half)
    pg2, qg2 = _sc_gather(p_tab, q_tab, d_idx[h:], s_idx[h:])
    msg2 = _edge_mlp(pg2, qg2, d2[h:], s2[h:], cp['ln_g'], cp['ln_b'],
                     cp['Wf'], cp['bf'], n_half)
    agg1 = _sc_scatter(msg1, d_idx[:h], zeros, n_half)
    agg2 = _sc_scatter(msg2, d_idx[h:], zeros, n_half)
    return _post(agg1, agg2, right, cp['pc_g'], cp['pc_b'],
                 cp['Wo1'][:, :EMBD], cp['Wo1'][:, EMBD:], cp['bo1'],
                 cp['Wo2'], cp['bo2'])


def kernel(params, constraint_features, edge_indices, edge_features,
           variable_features):
    p = params
    src = edge_indices[0]
    dst = edge_indices[1]
    e = src.shape[0]
    src2 = src.reshape(e, 1)
    dst2 = dst.reshape(e, 1)
    n_c = constraint_features.shape[0]
    edge_b = p['edge_ln_b'][0]

    half = n_c // 2 // SC_CORES
    pad = ((half + SC_SUBCORES) + 127) // 128 * 128
    zeros = jnp.zeros((pad, PK), F32)

    cf, cs = _embed(constraint_features, p['cons_ln_g'], p['cons_ln_b'],
                    p['cons_W1'], p['cons_b1'], p['cons_W2'], p['cons_b2'])
    cf = _se_apply(cf, cs, p['se_con_W1'], p['se_con_W2'])
    vf, vs = _embed(variable_features, p['var_ln_g'], p['var_ln_b'],
                    p['var_W1'], p['var_b1'], p['var_W2'], p['var_b2'])
    vf = _se_apply(vf, vs, p['se_var_W1'], p['se_var_W2'])

    vf = _conv(p['c2v'], cf, vf, src, dst, src2, dst2, edge_b, zeros)
    cf = _conv(p['v2c'], vf, cf, dst, src, dst2, src2, edge_b, zeros)
    vf = _conv(p['c2v2'], cf, vf, src, dst, src2, dst2, edge_b, zeros)
    cf = _conv(p['v2c2'], vf, cf, dst, src, dst2, src2, edge_b, zeros)

    con_out = _head(cf, p['con_W1'], p['con_b1'], p['con_W2'])
    var_out = _head(vf, p['varm_W1'], p['varm_b1'], p['varm_W2'])
    return jnp.squeeze(con_out, -1), jnp.squeeze(var_out, -1)


# speedup vs baseline: 2.1470x; 1.0851x over previous
import functools

import jax
import jax.numpy as jnp
from jax import lax
from jax.experimental import pallas as pl
from jax.experimental.pallas import tpu as pltpu
from jax.experimental.pallas import tpu_sc as plsc

F32 = jnp.float32
EMBD = 64
PK = 2 * EMBD
NBLK = 1000
EBLK = 2000
EPS = 1e-5

SC_CORES = 2
SC_SUBCORES = 16
NWORK = SC_CORES * SC_SUBCORES
GCHUNK = 128


def _mm(x, w):
    return lax.dot_general(x, w, (((1,), (1,)), ((), ())),
                           preferred_element_type=F32)


def _lnf(x, g, b):
    m = jnp.mean(x, axis=-1, keepdims=True)
    v = jnp.mean((x - m) ** 2, axis=-1, keepdims=True)
    return (x - m) / jnp.sqrt(v + EPS) * g + b


def _bspec(shape, imap):
    return pl.BlockSpec(shape, imap)


def _embed_body(xa_ref, xb_ref, g_ref, b_ref, w1_ref, b1_ref, w2_ref, b2_ref,
                out_ref, cs_ref):
    def half(x):
        h = _lnf(x, g_ref[...], b_ref[...])
        h = jnp.maximum(_mm(h, w1_ref[...]) + b1_ref[...], 0.0)
        return jnp.maximum(_mm(h, w2_ref[...]) + b2_ref[...], 0.0)

    ha = half(xa_ref[...])
    hb = half(xb_ref[...])
    out_ref[:, :EMBD] = ha
    out_ref[:, EMBD:] = hb

    @pl.when(pl.program_id(0) == 0)
    def _():
        cs_ref[...] = jnp.zeros_like(cs_ref)

    cs_ref[:, :EMBD] += jnp.sum(ha, axis=0, keepdims=True)
    cs_ref[:, EMBD:] += jnp.sum(hb, axis=0, keepdims=True)


def _embed(x, g, b, w1, b1, w2, b2):
    n, fin = x.shape
    n2 = n // 2
    grid = n2 // NBLK
    out, cs = pl.pallas_call(
        _embed_body,
        grid=(grid,),
        in_specs=[
            _bspec((NBLK, fin), lambda i: (i, 0)),
            _bspec((NBLK, fin), lambda i, g=grid: (i + g, 0)),
            _bspec((1, fin), lambda i: (0, 0)),
            _bspec((1, fin), lambda i: (0, 0)),
            _bspec((EMBD, fin), lambda i: (0, 0)),
            _bspec((1, EMBD), lambda i: (0, 0)),
            _bspec((EMBD, EMBD), lambda i: (0, 0)),
            _bspec((1, EMBD), lambda i: (0, 0)),
        ],
        out_specs=[
            _bspec((NBLK, PK), lambda i: (i, 0)),
            _bspec((1, PK), lambda i: (0, 0)),
        ],
        out_shape=[
            jax.ShapeDtypeStruct((n2, PK), F32),
            jax.ShapeDtypeStruct((1, PK), F32),
        ],
        compiler_params=pltpu.CompilerParams(
            dimension_semantics=("arbitrary",)),
    )(x, x, g.reshape(1, fin), b.reshape(1, fin), w1, b1.reshape(1, EMBD),
      w2, b2.reshape(1, EMBD))
    return out, cs


def _se_body(x_ref, cs_ref, w1_ref, w2_ref, out_ref, *, inv_n):
    mean = (cs_ref[:, :EMBD] + cs_ref[:, EMBD:]) * inv_n
    h = jnp.maximum(_mm(mean, w1_ref[...]), 0.0)
    w = jax.nn.sigmoid(_mm(h, w2_ref[...]))
    out_ref[:, :EMBD] = x_ref[:, :EMBD] * w
    out_ref[:, EMBD:] = x_ref[:, EMBD:] * w


def _se_apply(x, cs, w1, w2):
    n2 = x.shape[0]
    sq = w1.shape[0]
    return pl.pallas_call(
        functools.partial(_se_body, inv_n=0.5 / n2),
        grid=(n2 // NBLK,),
        in_specs=[
            _bspec((NBLK, PK), lambda i: (i, 0)),
            _bspec((1, PK), lambda i: (0, 0)),
            _bspec((sq, EMBD), lambda i: (0, 0)),
            _bspec((EMBD, sq), lambda i: (0, 0)),
        ],
        out_specs=_bspec((NBLK, PK), lambda i: (i, 0)),
        out_shape=jax.ShapeDtypeStruct((n2, PK), F32),
    )(x, cs, w1, w2)


def _pq_body(r_ref, l_ref, wl_ref, blc_ref, wr_ref, p_ref, q_ref):
    p_ref[:, :EMBD] = _mm(r_ref[:, :EMBD], wl_ref[...]) + blc_ref[...]
    p_ref[:, EMBD:] = _mm(r_ref[:, EMBD:], wl_ref[...]) + blc_ref[...]
    q_ref[:, :EMBD] = _mm(l_ref[:, :EMBD], wr_ref[...])
    q_ref[:, EMBD:] = _mm(l_ref[:, EMBD:], wr_ref[...])


def _pq(right, left, wl, blc, wr):
    n2 = right.shape[0]
    return pl.pallas_call(
        _pq_body,
        grid=(n2 // NBLK,),
        in_specs=[
            _bspec((NBLK, PK), lambda i: (i, 0)),
            _bspec((NBLK, PK), lambda i: (i, 0)),
            _bspec((EMBD, EMBD), lambda i: (0, 0)),
            _bspec((1, EMBD), lambda i: (0, 0)),
            _bspec((EMBD, EMBD), lambda i: (0, 0)),
        ],
        out_specs=[
            _bspec((NBLK, PK), lambda i: (i, 0)),
            _bspec((NBLK, PK), lambda i: (i, 0)),
        ],
        out_shape=[
            jax.ShapeDtypeStruct((n2, PK), F32),
            jax.ShapeDtypeStruct((n2, PK), F32),
        ],
    )(right, left, wl, blc, wr)


def _edge_body(pg_ref, qg_ref, d_ref, s_ref, g_ref, b_ref, wf_ref, bf_ref,
               msg_ref, *, n_half):
    pd = d_ref[...] >= n_half
    ps = s_ref[...] >= n_half
    p = jnp.where(pd, pg_ref[:, EMBD:], pg_ref[:, :EMBD])
    q = jnp.where(ps, qg_ref[:, EMBD:], qg_ref[:, :EMBD])
    h = _lnf(p + q, g_ref[...], b_ref[...])
    h = jnp.maximum(h, 0.0)
    m = _mm(h, wf_ref[...]) + bf_ref[...]
    msg_ref[:, :EMBD] = jnp.where(pd, 0.0, m)
    msg_ref[:, EMBD:] = jnp.where(pd, m, 0.0)


def _edge_mlp(pg, qg, d2, s2, g, b, wf, bf, n_half):
    e = pg.shape[0]
    return pl.pallas_call(
        functools.partial(_edge_body, n_half=n_half),
        grid=(e // EBLK,),
        in_specs=[
            _bspec((EBLK, PK), lambda i: (i, 0)),
            _bspec((EBLK, PK), lambda i: (i, 0)),
            _bspec((EBLK, 1), lambda i: (i, 0)),
            _bspec((EBLK, 1), lambda i: (i, 0)),
            _bspec((1, EMBD), lambda i: (0, 0)),
            _bspec((1, EMBD), lambda i: (0, 0)),
            _bspec((EMBD, EMBD), lambda i: (0, 0)),
            _bspec((1, EMBD), lambda i: (0, 0)),
        ],
        out_specs=_bspec((EBLK, PK), lambda i: (i, 0)),
        out_shape=jax.ShapeDtypeStruct((e, PK), F32),
    )(pg, qg, d2, s2, g.reshape(1, EMBD), b.reshape(1, EMBD), wf,
      bf.reshape(1, EMBD))


def _post_body(*refs, nagg):
    agg_refs = refs[:nagg]
    (r_ref, g_ref, b_ref, w1a_ref, w1b_ref, bo1_ref, w2_ref, bo2_ref,
     out_ref) = refs[nagg:]

    def half(sl):
        agg = agg_refs[0][:, sl]
        for a in agg_refs[1:]:
            agg = agg + a[:, sl]
        post = _lnf(agg, g_ref[...], b_ref[...])
        h = (_mm(post, w1a_ref[...]) + _mm(r_ref[:, sl], w1b_ref[...])
             + bo1_ref[...])
        h = jnp.maximum(h, 0.0)
        return _mm(h, w2_ref[...]) + bo2_ref[...]

    out_ref[:, :EMBD] = half(slice(0, EMBD))
    out_ref[:, EMBD:] = half(slice(EMBD, PK))


def _post(aggs, right, g, b, w1a, w1b, bo1, w2, bo2):
    n2 = aggs[0].shape[0]
    nagg = len(aggs)
    return pl.pallas_call(
        functools.partial(_post_body, nagg=nagg),
        grid=(n2 // NBLK,),
        in_specs=(
            [_bspec((NBLK, PK), lambda i: (i, 0))] * nagg
            + [
                _bspec((NBLK, PK), lambda i: (i, 0)),
                _bspec((1, EMBD), lambda i: (0, 0)),
                _bspec((1, EMBD), lambda i: (0, 0)),
                _bspec((EMBD, EMBD), lambda i: (0, 0)),
                _bspec((EMBD, EMBD), lambda i: (0, 0)),
                _bspec((1, EMBD), lambda i: (0, 0)),
                _bspec((EMBD, EMBD), lambda i: (0, 0)),
                _bspec((1, EMBD), lambda i: (0, 0)),
            ]
        ),
        out_specs=_bspec((NBLK, PK), lambda i: (i, 0)),
        out_shape=jax.ShapeDtypeStruct((n2, PK), F32),
    )(*aggs, right, g.reshape(1, EMBD), b.reshape(1, EMBD), w1a, w1b,
      bo1.reshape(1, EMBD), w2, bo2.reshape(1, EMBD))


def _head_body(x_ref, w1_ref, b1_ref, w2_ref, oa_ref, ob_ref):
    def half(sl):
        h = jnp.maximum(_mm(x_ref[:, sl], w1_ref[...]) + b1_ref[...], 0.0)
        return jax.nn.sigmoid(_mm(h, w2_ref[...]) * (1.0 / 0.6))

    oa_ref[...] = half(slice(0, EMBD))
    ob_ref[...] = half(slice(EMBD, PK))


def _head(x, w1, b1, w2):
    n2 = x.shape[0]
    oa, ob = pl.pallas_call(
        _head_body,
        grid=(n2 // NBLK,),
        in_specs=[
            _bspec((NBLK, PK), lambda i: (i, 0)),
            _bspec((EMBD, EMBD), lambda i: (0, 0)),
            _bspec((1, EMBD), lambda i: (0, 0)),
            _bspec((1, EMBD), lambda i: (0, 0)),
        ],
        out_specs=[
            _bspec((NBLK, 1), lambda i: (i, 0)),
            _bspec((NBLK, 1), lambda i: (i, 0)),
        ],
        out_shape=[
            jax.ShapeDtypeStruct((n2, 1), F32),
            jax.ShapeDtypeStruct((n2, 1), F32),
        ],
    )(x, w1, b1.reshape(1, EMBD), w2)
    return jnp.concatenate([oa, ob], axis=0)


def _sc_gather(p_tab, q_tab, d_idx, s_idx):
    e = d_idx.shape[0]
    n_half = p_tab.shape[0]
    per_w = e // NWORK
    iters = per_w // GCHUNK
    tail = per_w - iters * GCHUNK
    tail_p = (tail + 15) // 16 * 16
    mesh = plsc.VectorSubcoreMesh(core_axis_name="c", subcore_axis_name="s")

    @functools.partial(
        pl.kernel, mesh=mesh,
        out_type=(jax.ShapeDtypeStruct((e, PK), F32),
                  jax.ShapeDtypeStruct((e, PK), F32)),
        scratch_types=[
            pltpu.VMEM((GCHUNK,), jnp.int32),
            pltpu.VMEM((GCHUNK,), jnp.int32),
            pltpu.VMEM((GCHUNK, PK), F32),
            pltpu.VMEM((GCHUNK, PK), F32),
            pltpu.VMEM((max(tail_p, 16),), jnp.int32),
            pltpu.VMEM((max(tail_p, 16),), jnp.int32),
            pltpu.VMEM((max(tail_p, 16), PK), F32),
            pltpu.VMEM((max(tail_p, 16), PK), F32),
            pltpu.SemaphoreType.DMA,
            pltpu.SemaphoreType.DMA,
        ],
    )
    def k(p_hbm, q_hbm, d_hbm, s_hbm, pg_hbm, qg_hbm,
          di, si, pr, qr, dit, sit, prt, qrt, sem1, sem2):
        wid = lax.axis_index("s") * SC_CORES + lax.axis_index("c")
        base = wid * per_w

        def physify(ref, count):
            for kk in range((count + 15) // 16):
                v = ref[pl.ds(kk * 16, 16)]
                v = jnp.where(v >= n_half, v - n_half, v)
                if (kk + 1) * 16 > count:
                    lane = lax.iota(jnp.int32, 16)
                    v = jnp.where(lane < count - kk * 16, v, 0)
                ref[pl.ds(kk * 16, 16)] = v

        def step(off, dref, sref, prref, qrref, count):
            pltpu.sync_copy(d_hbm.at[pl.ds(off, count)],
                            dref.at[pl.ds(0, count)])
            pltpu.sync_copy(s_hbm.at[pl.ds(off, count)],
                            sref.at[pl.ds(0, count)])
            physify(dref, count)
            physify(sref, count)
            c1 = pltpu.async_copy(p_hbm.at[dref], prref, sem1)
            c2 = pltpu.async_copy(q_hbm.at[sref], qrref, sem2)
            c1.wait()
            c2.wait()
            pltpu.sync_copy(prref.at[pl.ds(0, count)],
                            pg_hbm.at[pl.ds(off, count)])
            pltpu.sync_copy(qrref.at[pl.ds(0, count)],
                            qg_hbm.at[pl.ds(off, count)])

        def body(j, carry):
            step(base + j * GCHUNK, di, si, pr, qr, GCHUNK)
            return carry

        lax.fori_loop(0, iters, body, 0)
        if tail:
            step(base + iters * GCHUNK, dit, sit, prt, qrt, tail)

    return k(p_tab, q_tab, d_idx, s_idx)


def _sc_scatter(msg, d_idx, zeros, n_half):
    e = d_idx.shape[0]
    half = n_half // SC_CORES
    pad = zeros.shape[0]
    rows_pt = pad // SC_SUBCORES
    chunk = 112
    per_t = e // SC_SUBCORES
    iters = per_t // chunk
    tail = per_t - iters * chunk
    depth = 2
    supers = iters // depth
    rem = iters - supers * depth
    mesh = plsc.VectorSubcoreMesh(core_axis_name="c", subcore_axis_name="s")

    @functools.partial(
        pl.kernel, mesh=mesh,
        out_type=jax.ShapeDtypeStruct((SC_CORES, pad, PK), F32),
        scratch_types=(
            [pltpu.VMEM((chunk,), jnp.int32)] * depth
            + [pltpu.VMEM((chunk, PK), F32)] * depth
            + [pltpu.SemaphoreType.DMA] * (2 * depth)
            + [pltpu.VMEM((max((tail + 15) // 16 * 16, 16),), jnp.int32),
               pltpu.VMEM_SHARED((pad, PK), F32)]
        ),
    )
    def k(msg_hbm, d_hbm, z_hbm, out_hbm, *refs):
        draws = refs[:depth]
        mbufs = refs[depth:2 * depth]
        lsems = refs[2 * depth:3 * depth]
        ssems = refs[3 * depth:4 * depth]
        drawt, acc = refs[4 * depth:]
        cid = lax.axis_index("c")
        sid = lax.axis_index("s")
        base = cid * half
        dump = half + sid

        zr = sid * rows_pt
        pltpu.sync_copy(z_hbm.at[pl.ds(zr, rows_pt)],
                        acc.at[pl.ds(zr, rows_pt)])
        plsc.subcore_barrier()

        def localize(ref, count):
            for kk in range((count + 15) // 16):
                v = ref[pl.ds(kk * 16, 16)]
                v = jnp.where(v >= n_half, v - n_half, v)
                l = v - base
                ok = (l >= 0) & (l < half)
                if (kk + 1) * 16 > count:
                    lane = lax.iota(jnp.int32, 16)
                    ok = ok & (lane < count - kk * 16)
                ref[pl.ds(kk * 16, 16)] = jnp.where(ok, l, dump)

        def pipe_group(off0):
            loads = []
            for b in range(depth):
                off = off0 + b * chunk
                c1 = pltpu.async_copy(d_hbm.at[pl.ds(off, chunk)],
                                      draws[b], lsems[b])
                c2 = pltpu.async_copy(msg_hbm.at[pl.ds(off, chunk)],
                                      mbufs[b], lsems[b])
                loads.append((c1, c2))
            scats = []
            for b in range(depth):
                loads[b][0].wait()
                loads[b][1].wait()
                localize(draws[b], chunk)
                scats.append(pltpu.async_copy(mbufs[b], acc.at[draws[b]],
                                              ssems[b], add=True))
            for s in scats:
                s.wait()

        def body(j, carry):
            pipe_group(sid * per_t + j * (depth * chunk))
            return carry

        lax.fori_loop(0, supers, body, 0)
        for r in range(rem):
            off = sid * per_t + (supers * depth + r) * chunk
            pltpu.sync_copy(d_hbm.at[pl.ds(off, chunk)], draws[0])
            pltpu.sync_copy(msg_hbm.at[pl.ds(off, chunk)], mbufs[0])
            localize(draws[0], chunk)
            pltpu.sync_copy(mbufs[0], acc.at[draws[0]], add=True)
        if tail:
            tail_p = (tail + 15) // 16 * 16
            off = sid * per_t + iters * chunk
            pltpu.sync_copy(d_hbm.at[pl.ds(off, tail)],
                            drawt.at[pl.ds(0, tail)])
            pltpu.sync_copy(msg_hbm.at[pl.ds(off, tail)],
                            mbufs[0].at[pl.ds(0, tail)])
            localize(drawt, tail)
            pltpu.sync_copy(mbufs[0].at[pl.ds(0, tail_p)],
                            acc.at[drawt], add=True)

        plsc.subcore_barrier()
        pltpu.sync_copy(acc.at[pl.ds(zr, rows_pt)],
                        out_hbm.at[cid, pl.ds(zr, rows_pt)])

    out = k(msg, d_idx, zeros)
    return jnp.concatenate([out[0, :half], out[1, :half]], axis=0)


def _conv(cp, left, right, s_idx, d_idx, s2, d2, edge_b, zeros):
    n_half = right.shape[0]
    e = d_idx.shape[0]
    step = ((e // 4) // 32000 * 32000) or ((e // 2) or e)
    bounds = []
    o = 0
    while o + step < e:
        bounds.append((o, step))
        o += step
        if len(bounds) == 3:
            break
    bounds.append((o, e - o))
    blc = (cp['bl'] + edge_b * cp['We'][:, 0]).reshape(1, EMBD)
    p_tab, q_tab = _pq(right, left, cp['Wl'], blc, cp['Wr'])
    aggs = []
    for o, sz in bounds:
        pg, qg = _sc_gather(p_tab, q_tab, d_idx[o:o + sz], s_idx[o:o + sz])
        msg = _edge_mlp(pg, qg, d2[o:o + sz], s2[o:o + sz], cp['ln_g'],
                        cp['ln_b'], cp['Wf'], cp['bf'], n_half)
        aggs.append(_sc_scatter(msg, d_idx[o:o + sz], zeros, n_half))
    return _post(aggs, right, cp['pc_g'], cp['pc_b'],
                 cp['Wo1'][:, :EMBD], cp['Wo1'][:, EMBD:], cp['bo1'],
                 cp['Wo2'], cp['bo2'])


def kernel(params, constraint_features, edge_indices, edge_features,
           variable_features):
    p = params
    src = edge_indices[0]
    dst = edge_indices[1]
    e = src.shape[0]
    src2 = src.reshape(e, 1)
    dst2 = dst.reshape(e, 1)
    n_c = constraint_features.shape[0]
    edge_b = p['edge_ln_b'][0]

    half = n_c // 2 // SC_CORES
    pad = ((half + SC_SUBCORES) + 127) // 128 * 128
    zeros = jnp.zeros((pad, PK), F32)

    cf, cs = _embed(constraint_features, p['cons_ln_g'], p['cons_ln_b'],
                    p['cons_W1'], p['cons_b1'], p['cons_W2'], p['cons_b2'])
    cf = _se_apply(cf, cs, p['se_con_W1'], p['se_con_W2'])
    vf, vs = _embed(variable_features, p['var_ln_g'], p['var_ln_b'],
                    p['var_W1'], p['var_b1'], p['var_W2'], p['var_b2'])
    vf = _se_apply(vf, vs, p['se_var_W1'], p['se_var_W2'])

    vf = _conv(p['c2v'], cf, vf, src, dst, src2, dst2, edge_b, zeros)
    cf = _conv(p['v2c'], vf, cf, dst, src, dst2, src2, edge_b, zeros)
    vf = _conv(p['c2v2'], cf, vf, src, dst, src2, dst2, edge_b, zeros)
    cf = _conv(p['v2c2'], vf, cf, dst, src, dst2, src2, edge_b, zeros)

    con_out = _head(cf, p['con_W1'], p['con_b1'], p['con_W2'])
    var_out = _head(vf, p['varm_W1'], p['varm_b1'], p['varm_W2'])
    return jnp.squeeze(con_out, -1), jnp.squeeze(var_out, -1)


# speedup vs baseline: 2.2958x; 1.0693x over previous
import functools

import jax
import jax.numpy as jnp
from jax import lax
from jax.experimental import pallas as pl
from jax.experimental.pallas import tpu as pltpu
from jax.experimental.pallas import tpu_sc as plsc

F32 = jnp.float32
EMBD = 64
PK = 2 * EMBD
NBLK = 1000
EBLK = 2000
EPS = 1e-5

SC_CORES = 2
SC_SUBCORES = 16
NWORK = SC_CORES * SC_SUBCORES
GCHUNK = 128


def _mm(x, w):
    return lax.dot_general(x, w, (((1,), (1,)), ((), ())),
                           preferred_element_type=F32)


def _lnf(x, g, b):
    m = jnp.mean(x, axis=-1, keepdims=True)
    v = jnp.mean((x - m) ** 2, axis=-1, keepdims=True)
    return (x - m) / jnp.sqrt(v + EPS) * g + b


def _bspec(shape, imap):
    return pl.BlockSpec(shape, imap)


def _embed_body(xa_ref, xb_ref, g_ref, b_ref, w1_ref, b1_ref, w2_ref, b2_ref,
                out_ref, cs_ref):
    def half(x):
        h = _lnf(x, g_ref[...], b_ref[...])
        h = jnp.maximum(_mm(h, w1_ref[...]) + b1_ref[...], 0.0)
        return jnp.maximum(_mm(h, w2_ref[...]) + b2_ref[...], 0.0)

    ha = half(xa_ref[...])
    hb = half(xb_ref[...])
    out_ref[:, :EMBD] = ha
    out_ref[:, EMBD:] = hb

    @pl.when(pl.program_id(0) == 0)
    def _():
        cs_ref[...] = jnp.zeros_like(cs_ref)

    cs_ref[:, :EMBD] += jnp.sum(ha, axis=0, keepdims=True)
    cs_ref[:, EMBD:] += jnp.sum(hb, axis=0, keepdims=True)


def _embed(x, g, b, w1, b1, w2, b2):
    n, fin = x.shape
    n2 = n // 2
    grid = n2 // NBLK
    out, cs = pl.pallas_call(
        _embed_body,
        grid=(grid,),
        in_specs=[
            _bspec((NBLK, fin), lambda i: (i, 0)),
            _bspec((NBLK, fin), lambda i, g=grid: (i + g, 0)),
            _bspec((1, fin), lambda i: (0, 0)),
            _bspec((1, fin), lambda i: (0, 0)),
            _bspec((EMBD, fin), lambda i: (0, 0)),
            _bspec((1, EMBD), lambda i: (0, 0)),
            _bspec((EMBD, EMBD), lambda i: (0, 0)),
            _bspec((1, EMBD), lambda i: (0, 0)),
        ],
        out_specs=[
            _bspec((NBLK, PK), lambda i: (i, 0)),
            _bspec((1, PK), lambda i: (0, 0)),
        ],
        out_shape=[
            jax.ShapeDtypeStruct((n2, PK), F32),
            jax.ShapeDtypeStruct((1, PK), F32),
        ],
        compiler_params=pltpu.CompilerParams(
            dimension_semantics=("arbitrary",)),
    )(x, x, g.reshape(1, fin), b.reshape(1, fin), w1, b1.reshape(1, EMBD),
      w2, b2.reshape(1, EMBD))
    return out, cs


def _se_body(x_ref, cs_ref, w1_ref, w2_ref, out_ref, *, inv_n):
    mean = (cs_ref[:, :EMBD] + cs_ref[:, EMBD:]) * inv_n
    h = jnp.maximum(_mm(mean, w1_ref[...]), 0.0)
    w = jax.nn.sigmoid(_mm(h, w2_ref[...]))
    out_ref[:, :EMBD] = x_ref[:, :EMBD] * w
    out_ref[:, EMBD:] = x_ref[:, EMBD:] * w


def _se_apply(x, cs, w1, w2):
    n2 = x.shape[0]
    sq = w1.shape[0]
    return pl.pallas_call(
        functools.partial(_se_body, inv_n=0.5 / n2),
        grid=(n2 // NBLK,),
        in_specs=[
            _bspec((NBLK, PK), lambda i: (i, 0)),
            _bspec((1, PK), lambda i: (0, 0)),
            _bspec((sq, EMBD), lambda i: (0, 0)),
            _bspec((EMBD, sq), lambda i: (0, 0)),
        ],
        out_specs=_bspec((NBLK, PK), lambda i: (i, 0)),
        out_shape=jax.ShapeDtypeStruct((n2, PK), F32),
    )(x, cs, w1, w2)


def _pq_body(r_ref, l_ref, wl_ref, blc_ref, wr_ref, p_ref, q_ref):
    hb = pl.program_id(0)
    x_r = jnp.where(hb == 0, r_ref[:, :EMBD], r_ref[:, EMBD:])
    x_l = jnp.where(hb == 0, l_ref[:, :EMBD], l_ref[:, EMBD:])
    pv = _mm(x_r, wl_ref[...]) + blc_ref[...]
    qv = _mm(x_l, wr_ref[...])
    p_ref[:, :EMBD] = pv
    p_ref[:, EMBD:] = pv
    q_ref[:, :EMBD] = qv
    q_ref[:, EMBD:] = qv


def _pq(right, left, wl, blc, wr):
    n2 = right.shape[0]
    g = n2 // NBLK
    return pl.pallas_call(
        _pq_body,
        grid=(2, g),
        in_specs=[
            _bspec((NBLK, PK), lambda h, i: (i, 0)),
            _bspec((NBLK, PK), lambda h, i: (i, 0)),
            _bspec((EMBD, EMBD), lambda h, i: (0, 0)),
            _bspec((1, EMBD), lambda h, i: (0, 0)),
            _bspec((EMBD, EMBD), lambda h, i: (0, 0)),
        ],
        out_specs=[
            _bspec((NBLK, PK), lambda h, i, g=g: (h * g + i, 0)),
            _bspec((NBLK, PK), lambda h, i, g=g: (h * g + i, 0)),
        ],
        out_shape=[
            jax.ShapeDtypeStruct((2 * n2, PK), F32),
            jax.ShapeDtypeStruct((2 * n2, PK), F32),
        ],
    )(right, left, wl, blc, wr)


def _edge_body(h_ref, d_ref, g_ref, b_ref, wf_ref, bf_ref,
               msg_ref, *, n_half):
    pd = d_ref[...] >= n_half
    h = _lnf(h_ref[...], g_ref[...], b_ref[...])
    h = jnp.maximum(h, 0.0)
    m = _mm(h, wf_ref[...]) + bf_ref[...]
    msg_ref[:, :EMBD] = jnp.where(pd, 0.0, m)
    msg_ref[:, EMBD:] = jnp.where(pd, m, 0.0)


def _edge_mlp(hpre, d2, g, b, wf, bf, n_half):
    e = hpre.shape[0]
    return pl.pallas_call(
        functools.partial(_edge_body, n_half=n_half),
        grid=(e // EBLK,),
        in_specs=[
            _bspec((EBLK, EMBD), lambda i: (i, 0)),
            _bspec((EBLK, 1), lambda i: (i, 0)),
            _bspec((1, EMBD), lambda i: (0, 0)),
            _bspec((1, EMBD), lambda i: (0, 0)),
            _bspec((EMBD, EMBD), lambda i: (0, 0)),
            _bspec((1, EMBD), lambda i: (0, 0)),
        ],
        out_specs=_bspec((EBLK, PK), lambda i: (i, 0)),
        out_shape=jax.ShapeDtypeStruct((e, PK), F32),
    )(hpre, d2, g.reshape(1, EMBD), b.reshape(1, EMBD), wf,
      bf.reshape(1, EMBD))


def _post_body(*refs, nagg):
    agg_refs = refs[:nagg]
    (r_ref, g_ref, b_ref, w1a_ref, w1b_ref, bo1_ref, w2_ref, bo2_ref,
     out_ref) = refs[nagg:]

    def half(sl):
        agg = agg_refs[0][:, sl]
        for a in agg_refs[1:]:
            agg = agg + a[:, sl]
        post = _lnf(agg, g_ref[...], b_ref[...])
        h = (_mm(post, w1a_ref[...]) + _mm(r_ref[:, sl], w1b_ref[...])
             + bo1_ref[...])
        h = jnp.maximum(h, 0.0)
        return _mm(h, w2_ref[...]) + bo2_ref[...]

    out_ref[:, :EMBD] = half(slice(0, EMBD))
    out_ref[:, EMBD:] = half(slice(EMBD, PK))


def _post(aggs, right, g, b, w1a, w1b, bo1, w2, bo2):
    n2 = aggs[0].shape[0]
    nagg = len(aggs)
    return pl.pallas_call(
        functools.partial(_post_body, nagg=nagg),
        grid=(n2 // NBLK,),
        in_specs=(
            [_bspec((NBLK, PK), lambda i: (i, 0))] * nagg
            + [
                _bspec((NBLK, PK), lambda i: (i, 0)),
                _bspec((1, EMBD), lambda i: (0, 0)),
                _bspec((1, EMBD), lambda i: (0, 0)),
                _bspec((EMBD, EMBD), lambda i: (0, 0)),
                _bspec((EMBD, EMBD), lambda i: (0, 0)),
                _bspec((1, EMBD), lambda i: (0, 0)),
                _bspec((EMBD, EMBD), lambda i: (0, 0)),
                _bspec((1, EMBD), lambda i: (0, 0)),
            ]
        ),
        out_specs=_bspec((NBLK, PK), lambda i: (i, 0)),
        out_shape=jax.ShapeDtypeStruct((n2, PK), F32),
    )(*aggs, right, g.reshape(1, EMBD), b.reshape(1, EMBD), w1a, w1b,
      bo1.reshape(1, EMBD), w2, bo2.reshape(1, EMBD))


def _head_body(x_ref, w1_ref, b1_ref, w2_ref, oa_ref, ob_ref):
    def half(sl):
        h = jnp.maximum(_mm(x_ref[:, sl], w1_ref[...]) + b1_ref[...], 0.0)
        return jax.nn.sigmoid(_mm(h, w2_ref[...]) * (1.0 / 0.6))

    oa_ref[...] = half(slice(0, EMBD))
    ob_ref[...] = half(slice(EMBD, PK))


def _head(x, w1, b1, w2):
    n2 = x.shape[0]
    oa, ob = pl.pallas_call(
        _head_body,
        grid=(n2 // NBLK,),
        in_specs=[
            _bspec((NBLK, PK), lambda i: (i, 0)),
            _bspec((EMBD, EMBD), lambda i: (0, 0)),
            _bspec((1, EMBD), lambda i: (0, 0)),
            _bspec((1, EMBD), lambda i: (0, 0)),
        ],
        out_specs=[
            _bspec((NBLK, 1), lambda i: (i, 0)),
            _bspec((NBLK, 1), lambda i: (i, 0)),
        ],
        out_shape=[
            jax.ShapeDtypeStruct((n2, 1), F32),
            jax.ShapeDtypeStruct((n2, 1), F32),
        ],
    )(x, w1, b1.reshape(1, EMBD), w2)
    return jnp.concatenate([oa, ob], axis=0)


def _sc_gather(p_tab, q_tab, d_idx, s_idx):
    e = d_idx.shape[0]
    per_w = e // NWORK
    iters = per_w // GCHUNK
    tail = per_w - iters * GCHUNK
    tail_p = (tail + 15) // 16 * 16
    depth = 2
    supers = iters // depth
    rem = iters - supers * depth
    mesh = plsc.VectorSubcoreMesh(core_axis_name="c", subcore_axis_name="s")

    @functools.partial(
        pl.kernel, mesh=mesh,
        out_type=jax.ShapeDtypeStruct((e, EMBD), F32),
        scratch_types=(
            [pltpu.VMEM((GCHUNK,), jnp.int32)] * (2 * depth)
            + [pltpu.VMEM((GCHUNK, PK), F32)] * (2 * depth)
            + [pltpu.VMEM((GCHUNK, EMBD), F32)] * depth
            + [pltpu.SemaphoreType.DMA] * (3 * depth)
            + [pltpu.VMEM((max(tail_p, 16),), jnp.int32)] * 2
        ),
    )
    def k(p_hbm, q_hbm, d_hbm, s_hbm, hp_hbm, *refs):
        dis = refs[:depth]
        sis = refs[depth:2 * depth]
        prs = refs[2 * depth:3 * depth]
        qrs = refs[3 * depth:4 * depth]
        hbs = refs[4 * depth:5 * depth]
        isems = refs[5 * depth:6 * depth]
        gsems = refs[6 * depth:7 * depth]
        wsems = refs[7 * depth:8 * depth]
        dit, sit = refs[8 * depth:]
        wid = lax.axis_index("s") * SC_CORES + lax.axis_index("c")
        base = wid * per_w

        def combine(prref, qrref, hbref, count):
            for r in range(count):
                for kk in range(EMBD // 16):
                    sl = pl.ds(kk * 16, 16)
                    hbref[r, sl] = prref[r, sl] + qrref[r, sl]

        def pipe_group(off0):
            loads = []
            for b in range(depth):
                off = off0 + b * GCHUNK
                l1 = pltpu.async_copy(d_hbm.at[pl.ds(off, GCHUNK)],
                                      dis[b], isems[b])
                l2 = pltpu.async_copy(s_hbm.at[pl.ds(off, GCHUNK)],
                                      sis[b], isems[b])
                loads.append((l1, l2))
            gats = []
            for b in range(depth):
                loads[b][0].wait()
                loads[b][1].wait()
                g1 = pltpu.async_copy(p_hbm.at[dis[b]], prs[b], gsems[b])
                g2 = pltpu.async_copy(q_hbm.at[sis[b]], qrs[b], gsems[b])
                gats.append((g1, g2))
            writes = []
            for b in range(depth):
                off = off0 + b * GCHUNK
                gats[b][0].wait()
                gats[b][1].wait()
                combine(prs[b], qrs[b], hbs[b], GCHUNK)
                writes.append(pltpu.async_copy(
                    hbs[b], hp_hbm.at[pl.ds(off, GCHUNK)], wsems[b]))
            for w in writes:
                w.wait()

        def body(j, carry):
            pipe_group(base + j * (depth * GCHUNK))
            return carry

        lax.fori_loop(0, supers, body, 0)
        for r in range(rem):
            off = base + (supers * depth + r) * GCHUNK
            pltpu.sync_copy(d_hbm.at[pl.ds(off, GCHUNK)], dis[0])
            pltpu.sync_copy(s_hbm.at[pl.ds(off, GCHUNK)], sis[0])
            c1 = pltpu.async_copy(p_hbm.at[dis[0]], prs[0], gsems[0])
            c2 = pltpu.async_copy(q_hbm.at[sis[0]], qrs[0], gsems[0])
            c1.wait()
            c2.wait()
            combine(prs[0], qrs[0], hbs[0], GCHUNK)
            pltpu.sync_copy(hbs[0], hp_hbm.at[pl.ds(off, GCHUNK)])
        if tail:
            off = base + iters * GCHUNK
            pltpu.sync_copy(d_hbm.at[pl.ds(off, tail)],
                            dit.at[pl.ds(0, tail)])
            pltpu.sync_copy(s_hbm.at[pl.ds(off, tail)],
                            sit.at[pl.ds(0, tail)])
            for ref in (dit, sit):
                for kk in range((tail + 15) // 16):
                    v = ref[pl.ds(kk * 16, 16)]
                    if (kk + 1) * 16 > tail:
                        lane = lax.iota(jnp.int32, 16)
                        v = jnp.where(lane < tail - kk * 16, v, 0)
                    ref[pl.ds(kk * 16, 16)] = v
            c1 = pltpu.async_copy(p_hbm.at[dit],
                                  prs[0].at[pl.ds(0, max(tail_p, 16))],
                                  gsems[0])
            c2 = pltpu.async_copy(q_hbm.at[sit],
                                  qrs[0].at[pl.ds(0, max(tail_p, 16))],
                                  gsems[0])
            c1.wait()
            c2.wait()
            combine(prs[0], qrs[0], hbs[0], tail)
            pltpu.sync_copy(hbs[0].at[pl.ds(0, tail)],
                            hp_hbm.at[pl.ds(off, tail)])

    return k(p_tab, q_tab, d_idx, s_idx)


def _sc_scatter(msg, d_idx, zeros, n_half):
    e = d_idx.shape[0]
    half = n_half // SC_CORES
    pad = zeros.shape[0]
    rows_pt = pad // SC_SUBCORES
    chunk = 112
    per_t = e // SC_SUBCORES
    iters = per_t // chunk
    tail = per_t - iters * chunk
    depth = 2
    supers = iters // depth
    rem = iters - supers * depth
    mesh = plsc.VectorSubcoreMesh(core_axis_name="c", subcore_axis_name="s")

    @functools.partial(
        pl.kernel, mesh=mesh,
        out_type=jax.ShapeDtypeStruct((SC_CORES, pad, PK), F32),
        scratch_types=(
            [pltpu.VMEM((chunk,), jnp.int32)] * depth
            + [pltpu.VMEM((chunk, PK), F32)] * depth
            + [pltpu.SemaphoreType.DMA] * (2 * depth)
            + [pltpu.VMEM((max((tail + 15) // 16 * 16, 16),), jnp.int32),
               pltpu.VMEM_SHARED((pad, PK), F32)]
        ),
    )
    def k(msg_hbm, d_hbm, z_hbm, out_hbm, *refs):
        draws = refs[:depth]
        mbufs = refs[depth:2 * depth]
        lsems = refs[2 * depth:3 * depth]
        ssems = refs[3 * depth:4 * depth]
        drawt, acc = refs[4 * depth:]
        cid = lax.axis_index("c")
        sid = lax.axis_index("s")
        base = cid * half
        dump = half + sid

        zr = sid * rows_pt
        pltpu.sync_copy(z_hbm.at[pl.ds(zr, rows_pt)],
                        acc.at[pl.ds(zr, rows_pt)])
        plsc.subcore_barrier()

        def localize(ref, count):
            for kk in range((count + 15) // 16):
                v = ref[pl.ds(kk * 16, 16)]
                v = jnp.where(v >= n_half, v - n_half, v)
                l = v - base
                ok = (l >= 0) & (l < half)
                if (kk + 1) * 16 > count:
                    lane = lax.iota(jnp.int32, 16)
                    ok = ok & (lane < count - kk * 16)
                ref[pl.ds(kk * 16, 16)] = jnp.where(ok, l, dump)

        def pipe_group(off0):
            loads = []
            for b in range(depth):
                off = off0 + b * chunk
                c1 = pltpu.async_copy(d_hbm.at[pl.ds(off, chunk)],
                                      draws[b], lsems[b])
                c2 = pltpu.async_copy(msg_hbm.at[pl.ds(off, chunk)],
                                      mbufs[b], lsems[b])
                loads.append((c1, c2))
            scats = []
            for b in range(depth):
                loads[b][0].wait()
                loads[b][1].wait()
                localize(draws[b], chunk)
                scats.append(pltpu.async_copy(mbufs[b], acc.at[draws[b]],
                                              ssems[b], add=True))
            for s in scats:
                s.wait()

        def body(j, carry):
            pipe_group(sid * per_t + j * (depth * chunk))
            return carry

        lax.fori_loop(0, supers, body, 0)
        for r in range(rem):
            off = sid * per_t + (supers * depth + r) * chunk
            pltpu.sync_copy(d_hbm.at[pl.ds(off, chunk)], draws[0])
            pltpu.sync_copy(msg_hbm.at[pl.ds(off, chunk)], mbufs[0])
            localize(draws[0], chunk)
            pltpu.sync_copy(mbufs[0], acc.at[draws[0]], add=True)
        if tail:
            tail_p = (tail + 15) // 16 * 16
            off = sid * per_t + iters * chunk
            pltpu.sync_copy(d_hbm.at[pl.ds(off, tail)],
                            drawt.at[pl.ds(0, tail)])
            pltpu.sync_copy(msg_hbm.at[pl.ds(off, tail)],
                            mbufs[0].at[pl.ds(0, tail)])
            localize(drawt, tail)
            pltpu.sync_copy(mbufs[0].at[pl.ds(0, tail_p)],
                            acc.at[drawt], add=True)

        plsc.subcore_barrier()
        pltpu.sync_copy(acc.at[pl.ds(zr, rows_pt)],
                        out_hbm.at[cid, pl.ds(zr, rows_pt)])

    out = k(msg, d_idx, zeros)
    return jnp.concatenate([out[0, :half], out[1, :half]], axis=0)


def _conv(cp, left, right, s_idx, d_idx, s2, d2, edge_b, zeros):
    n_half = right.shape[0]
    e = d_idx.shape[0]
    step = ((e // 4) // 32000 * 32000) or ((e // 2) or e)
    bounds = []
    o = 0
    while o + step < e:
        bounds.append((o, step))
        o += step
        if len(bounds) == 3:
            break
    bounds.append((o, e - o))
    blc = (cp['bl'] + edge_b * cp['We'][:, 0]).reshape(1, EMBD)
    p_tab, q_tab = _pq(right, left, cp['Wl'], blc, cp['Wr'])
    aggs = []
    for o, sz in bounds:
        hpre = _sc_gather(p_tab, q_tab, d_idx[o:o + sz], s_idx[o:o + sz])
        msg = _edge_mlp(hpre, d2[o:o + sz], cp['ln_g'],
                        cp['ln_b'], cp['Wf'], cp['bf'], n_half)
        aggs.append(_sc_scatter(msg, d_idx[o:o + sz], zeros, n_half))
    return _post(aggs, right, cp['pc_g'], cp['pc_b'],
                 cp['Wo1'][:, :EMBD], cp['Wo1'][:, EMBD:], cp['bo1'],
                 cp['Wo2'], cp['bo2'])


def kernel(params, constraint_features, edge_indices, edge_features,
           variable_features):
    p = params
    src = edge_indices[0]
    dst = edge_indices[1]
    e = src.shape[0]
    src2 = src.reshape(e, 1)
    dst2 = dst.reshape(e, 1)
    n_c = constraint_features.shape[0]
    edge_b = p['edge_ln_b'][0]

    half = n_c // 2 // SC_CORES
    pad = ((half + SC_SUBCORES) + 127) // 128 * 128
    zeros = jnp.zeros((pad, PK), F32)

    cf, cs = _embed(constraint_features, p['cons_ln_g'], p['cons_ln_b'],
                    p['cons_W1'], p['cons_b1'], p['cons_W2'], p['cons_b2'])
    cf = _se_apply(cf, cs, p['se_con_W1'], p['se_con_W2'])
    vf, vs = _embed(variable_features, p['var_ln_g'], p['var_ln_b'],
                    p['var_W1'], p['var_b1'], p['var_W2'], p['var_b2'])
    vf = _se_apply(vf, vs, p['se_var_W1'], p['se_var_W2'])

    vf = _conv(p['c2v'], cf, vf, src, dst, src2, dst2, edge_b, zeros)
    cf = _conv(p['v2c'], vf, cf, dst, src, dst2, src2, edge_b, zeros)
    vf = _conv(p['c2v2'], cf, vf, src, dst, src2, dst2, edge_b, zeros)
    cf = _conv(p['v2c2'], vf, cf, dst, src, dst2, src2, edge_b, zeros)

    con_out = _head(cf, p['con_W1'], p['con_b1'], p['con_W2'])
    var_out = _head(vf, p['varm_W1'], p['varm_b1'], p['varm_W2'])
    return jnp.squeeze(con_out, -1), jnp.squeeze(var_out, -1)


# speedup vs baseline: 2.6937x; 1.1733x over previous
import functools

import jax
import jax.numpy as jnp
from jax import lax
from jax.experimental import pallas as pl
from jax.experimental.pallas import tpu as pltpu
from jax.experimental.pallas import tpu_sc as plsc

F32 = jnp.float32
EMBD = 64
PK = 2 * EMBD
NBLK = 1000
EBLK = 2000
EPS = 1e-5

SC_CORES = 2
SC_SUBCORES = 16
NWORK = SC_CORES * SC_SUBCORES
GCHUNK = 128


def _mm(x, w):
    return lax.dot_general(x, w, (((1,), (1,)), ((), ())),
                           preferred_element_type=F32)


def _lnf(x, g, b):
    m = jnp.mean(x, axis=-1, keepdims=True)
    v = jnp.mean((x - m) ** 2, axis=-1, keepdims=True)
    return (x - m) / jnp.sqrt(v + EPS) * g + b


def _bspec(shape, imap):
    return pl.BlockSpec(shape, imap)


def _embed_body(xa_ref, xb_ref, g_ref, b_ref, w1_ref, b1_ref, w2_ref, b2_ref,
                out_ref, cs_ref):
    def half(x):
        h = _lnf(x, g_ref[...], b_ref[...])
        h = jnp.maximum(_mm(h, w1_ref[...]) + b1_ref[...], 0.0)
        return jnp.maximum(_mm(h, w2_ref[...]) + b2_ref[...], 0.0)

    ha = half(xa_ref[...])
    hb = half(xb_ref[...])
    out_ref[:, :EMBD] = ha
    out_ref[:, EMBD:] = hb

    @pl.when(pl.program_id(0) == 0)
    def _():
        cs_ref[...] = jnp.zeros_like(cs_ref)

    cs_ref[:, :EMBD] += jnp.sum(ha, axis=0, keepdims=True)
    cs_ref[:, EMBD:] += jnp.sum(hb, axis=0, keepdims=True)


def _embed(x, g, b, w1, b1, w2, b2):
    n, fin = x.shape
    n2 = n // 2
    grid = n2 // NBLK
    out, cs = pl.pallas_call(
        _embed_body,
        grid=(grid,),
        in_specs=[
            _bspec((NBLK, fin), lambda i: (i, 0)),
            _bspec((NBLK, fin), lambda i, g=grid: (i + g, 0)),
            _bspec((1, fin), lambda i: (0, 0)),
            _bspec((1, fin), lambda i: (0, 0)),
            _bspec((EMBD, fin), lambda i: (0, 0)),
            _bspec((1, EMBD), lambda i: (0, 0)),
            _bspec((EMBD, EMBD), lambda i: (0, 0)),
            _bspec((1, EMBD), lambda i: (0, 0)),
        ],
        out_specs=[
            _bspec((NBLK, PK), lambda i: (i, 0)),
            _bspec((1, PK), lambda i: (0, 0)),
        ],
        out_shape=[
            jax.ShapeDtypeStruct((n2, PK), F32),
            jax.ShapeDtypeStruct((1, PK), F32),
        ],
        compiler_params=pltpu.CompilerParams(
            dimension_semantics=("arbitrary",)),
    )(x, x, g.reshape(1, fin), b.reshape(1, fin), w1, b1.reshape(1, EMBD),
      w2, b2.reshape(1, EMBD))
    return out, cs


def _se_body(x_ref, cs_ref, w1_ref, w2_ref, out_ref, *, inv_n):
    mean = (cs_ref[:, :EMBD] + cs_ref[:, EMBD:]) * inv_n
    h = jnp.maximum(_mm(mean, w1_ref[...]), 0.0)
    w = jax.nn.sigmoid(_mm(h, w2_ref[...]))
    out_ref[:, :EMBD] = x_ref[:, :EMBD] * w
    out_ref[:, EMBD:] = x_ref[:, EMBD:] * w


def _se_apply(x, cs, w1, w2):
    n2 = x.shape[0]
    sq = w1.shape[0]
    return pl.pallas_call(
        functools.partial(_se_body, inv_n=0.5 / n2),
        grid=(n2 // NBLK,),
        in_specs=[
            _bspec((NBLK, PK), lambda i: (i, 0)),
            _bspec((1, PK), lambda i: (0, 0)),
            _bspec((sq, EMBD), lambda i: (0, 0)),
            _bspec((EMBD, sq), lambda i: (0, 0)),
        ],
        out_specs=_bspec((NBLK, PK), lambda i: (i, 0)),
        out_shape=jax.ShapeDtypeStruct((n2, PK), F32),
    )(x, cs, w1, w2)


def _pq_body(r_ref, l_ref, wl_ref, blc_ref, wr_ref, p_ref, q_ref):
    hb = pl.program_id(0)
    x_r = jnp.where(hb == 0, r_ref[:, :EMBD], r_ref[:, EMBD:])
    x_l = jnp.where(hb == 0, l_ref[:, :EMBD], l_ref[:, EMBD:])
    pv = _mm(x_r, wl_ref[...]) + blc_ref[...]
    qv = _mm(x_l, wr_ref[...])
    p_ref[:, :EMBD] = pv
    p_ref[:, EMBD:] = pv
    q_ref[:, :EMBD] = qv
    q_ref[:, EMBD:] = qv


def _pq(right, left, wl, blc, wr):
    n2 = right.shape[0]
    g = n2 // NBLK
    return pl.pallas_call(
        _pq_body,
        grid=(2, g),
        in_specs=[
            _bspec((NBLK, PK), lambda h, i: (i, 0)),
            _bspec((NBLK, PK), lambda h, i: (i, 0)),
            _bspec((EMBD, EMBD), lambda h, i: (0, 0)),
            _bspec((1, EMBD), lambda h, i: (0, 0)),
            _bspec((EMBD, EMBD), lambda h, i: (0, 0)),
        ],
        out_specs=[
            _bspec((NBLK, PK), lambda h, i, g=g: (h * g + i, 0)),
            _bspec((NBLK, PK), lambda h, i, g=g: (h * g + i, 0)),
        ],
        out_shape=[
            jax.ShapeDtypeStruct((2 * n2, PK), F32),
            jax.ShapeDtypeStruct((2 * n2, PK), F32),
        ],
    )(right, left, wl, blc, wr)


def _edge_body(h_ref, d_ref, g_ref, b_ref, wf_ref, bf_ref,
               msg_ref, *, n_half):
    pd = d_ref[...] >= n_half
    h = _lnf(h_ref[...], g_ref[...], b_ref[...])
    h = jnp.maximum(h, 0.0)
    m = _mm(h, wf_ref[...]) + bf_ref[...]
    msg_ref[:, :EMBD] = jnp.where(pd, 0.0, m)
    msg_ref[:, EMBD:] = jnp.where(pd, m, 0.0)


def _edge_mlp(hpre, d2, g, b, wf, bf, n_half):
    e = hpre.shape[0]
    return pl.pallas_call(
        functools.partial(_edge_body, n_half=n_half),
        grid=(e // EBLK,),
        in_specs=[
            _bspec((EBLK, EMBD), lambda i: (i, 0)),
            _bspec((EBLK, 1), lambda i: (i, 0)),
            _bspec((1, EMBD), lambda i: (0, 0)),
            _bspec((1, EMBD), lambda i: (0, 0)),
            _bspec((EMBD, EMBD), lambda i: (0, 0)),
            _bspec((1, EMBD), lambda i: (0, 0)),
        ],
        out_specs=_bspec((EBLK, PK), lambda i: (i, 0)),
        out_shape=jax.ShapeDtypeStruct((e, PK), F32),
    )(hpre, d2, g.reshape(1, EMBD), b.reshape(1, EMBD), wf,
      bf.reshape(1, EMBD))


def _post_body(*refs, nagg):
    agg_refs = refs[:nagg]
    (r_ref, g_ref, b_ref, w1a_ref, w1b_ref, bo1_ref, w2_ref, bo2_ref,
     out_ref) = refs[nagg:]

    def half(sl):
        agg = agg_refs[0][:, sl]
        for a in agg_refs[1:]:
            agg = agg + a[:, sl]
        post = _lnf(agg, g_ref[...], b_ref[...])
        h = (_mm(post, w1a_ref[...]) + _mm(r_ref[:, sl], w1b_ref[...])
             + bo1_ref[...])
        h = jnp.maximum(h, 0.0)
        return _mm(h, w2_ref[...]) + bo2_ref[...]

    out_ref[:, :EMBD] = half(slice(0, EMBD))
    out_ref[:, EMBD:] = half(slice(EMBD, PK))


def _post(aggs, right, g, b, w1a, w1b, bo1, w2, bo2):
    n2 = aggs[0].shape[0]
    nagg = len(aggs)
    return pl.pallas_call(
        functools.partial(_post_body, nagg=nagg),
        grid=(n2 // NBLK,),
        in_specs=(
            [_bspec((NBLK, PK), lambda i: (i, 0))] * nagg
            + [
                _bspec((NBLK, PK), lambda i: (i, 0)),
                _bspec((1, EMBD), lambda i: (0, 0)),
                _bspec((1, EMBD), lambda i: (0, 0)),
                _bspec((EMBD, EMBD), lambda i: (0, 0)),
                _bspec((EMBD, EMBD), lambda i: (0, 0)),
                _bspec((1, EMBD), lambda i: (0, 0)),
                _bspec((EMBD, EMBD), lambda i: (0, 0)),
                _bspec((1, EMBD), lambda i: (0, 0)),
            ]
        ),
        out_specs=_bspec((NBLK, PK), lambda i: (i, 0)),
        out_shape=jax.ShapeDtypeStruct((n2, PK), F32),
    )(*aggs, right, g.reshape(1, EMBD), b.reshape(1, EMBD), w1a, w1b,
      bo1.reshape(1, EMBD), w2, bo2.reshape(1, EMBD))


def _head_body(x_ref, w1_ref, b1_ref, w2_ref, oa_ref, ob_ref):
    def half(sl):
        h = jnp.maximum(_mm(x_ref[:, sl], w1_ref[...]) + b1_ref[...], 0.0)
        return jax.nn.sigmoid(_mm(h, w2_ref[...]) * (1.0 / 0.6))

    oa_ref[...] = half(slice(0, EMBD))
    ob_ref[...] = half(slice(EMBD, PK))


def _head(x, w1, b1, w2):
    n2 = x.shape[0]
    oa, ob = pl.pallas_call(
        _head_body,
        grid=(n2 // NBLK,),
        in_specs=[
            _bspec((NBLK, PK), lambda i: (i, 0)),
            _bspec((EMBD, EMBD), lambda i: (0, 0)),
            _bspec((1, EMBD), lambda i: (0, 0)),
            _bspec((1, EMBD), lambda i: (0, 0)),
        ],
        out_specs=[
            _bspec((NBLK, 1), lambda i: (i, 0)),
            _bspec((NBLK, 1), lambda i: (i, 0)),
        ],
        out_shape=[
            jax.ShapeDtypeStruct((n2, 1), F32),
            jax.ShapeDtypeStruct((n2, 1), F32),
        ],
    )(x, w1, b1.reshape(1, EMBD), w2)
    return jnp.concatenate([oa, ob], axis=0)


def _sc_gather(p_tab, q_tab, d_idx, s_idx):
    e = d_idx.shape[0]
    per_w = e // NWORK
    iters = per_w // GCHUNK
    tail = per_w - iters * GCHUNK
    tail_p = (tail + 15) // 16 * 16
    depth = 2
    supers = iters // depth
    rem = iters - supers * depth
    mesh = plsc.VectorSubcoreMesh(core_axis_name="c", subcore_axis_name="s")

    @functools.partial(
        pl.kernel, mesh=mesh,
        out_type=jax.ShapeDtypeStruct((e, EMBD), F32),
        scratch_types=(
            [pltpu.VMEM((GCHUNK,), jnp.int32)] * (2 * depth)
            + [pltpu.VMEM((GCHUNK, PK), F32)] * (2 * depth)
            + [pltpu.VMEM((GCHUNK, EMBD), F32)] * depth
            + [pltpu.SemaphoreType.DMA] * (3 * depth)
            + [pltpu.VMEM((max(tail_p, 16),), jnp.int32)] * 2
        ),
    )
    def k(p_hbm, q_hbm, d_hbm, s_hbm, hp_hbm, *refs):
        dis = refs[:depth]
        sis = refs[depth:2 * depth]
        prs = refs[2 * depth:3 * depth]
        qrs = refs[3 * depth:4 * depth]
        hbs = refs[4 * depth:5 * depth]
        isems = refs[5 * depth:6 * depth]
        gsems = refs[6 * depth:7 * depth]
        wsems = refs[7 * depth:8 * depth]
        dit, sit = refs[8 * depth:]
        wid = lax.axis_index("s") * SC_CORES + lax.axis_index("c")
        base = wid * per_w

        def combine(prref, qrref, hbref, count):
            def cb(r, c):
                for kk in range(EMBD // 16):
                    sl = pl.ds(kk * 16, 16)
                    hbref[r, sl] = prref[r, sl] + qrref[r, sl]
                return c

            lax.fori_loop(0, count, cb, 0)

        def pipe_group(off0):
            loads = []
            for b in range(depth):
                off = off0 + b * GCHUNK
                l1 = pltpu.async_copy(d_hbm.at[pl.ds(off, GCHUNK)],
                                      dis[b], isems[b])
                l2 = pltpu.async_copy(s_hbm.at[pl.ds(off, GCHUNK)],
                                      sis[b], isems[b])
                loads.append((l1, l2))
            gats = []
            for b in range(depth):
                loads[b][0].wait()
                loads[b][1].wait()
                g1 = pltpu.async_copy(p_hbm.at[dis[b]], prs[b], gsems[b])
                g2 = pltpu.async_copy(q_hbm.at[sis[b]], qrs[b], gsems[b])
                gats.append((g1, g2))
            writes = []
            for b in range(depth):
                off = off0 + b * GCHUNK
                gats[b][0].wait()
                gats[b][1].wait()
                combine(prs[b], qrs[b], hbs[b], GCHUNK)
                writes.append(pltpu.async_copy(
                    hbs[b], hp_hbm.at[pl.ds(off, GCHUNK)], wsems[b]))
            for w in writes:
                w.wait()

        def body(j, carry):
            pipe_group(base + j * (depth * GCHUNK))
            return carry

        lax.fori_loop(0, supers, body, 0)
        for r in range(rem):
            off = base + (supers * depth + r) * GCHUNK
            pltpu.sync_copy(d_hbm.at[pl.ds(off, GCHUNK)], dis[0])
            pltpu.sync_copy(s_hbm.at[pl.ds(off, GCHUNK)], sis[0])
            c1 = pltpu.async_copy(p_hbm.at[dis[0]], prs[0], gsems[0])
            c2 = pltpu.async_copy(q_hbm.at[sis[0]], qrs[0], gsems[0])
            c1.wait()
            c2.wait()
            combine(prs[0], qrs[0], hbs[0], GCHUNK)
            pltpu.sync_copy(hbs[0], hp_hbm.at[pl.ds(off, GCHUNK)])
        if tail:
            off = base + iters * GCHUNK
            pltpu.sync_copy(d_hbm.at[pl.ds(off, tail)],
                            dit.at[pl.ds(0, tail)])
            pltpu.sync_copy(s_hbm.at[pl.ds(off, tail)],
                            sit.at[pl.ds(0, tail)])
            for ref in (dit, sit):
                for kk in range((tail + 15) // 16):
                    v = ref[pl.ds(kk * 16, 16)]
                    if (kk + 1) * 16 > tail:
                        lane = lax.iota(jnp.int32, 16)
                        v = jnp.where(lane < tail - kk * 16, v, 0)
                    ref[pl.ds(kk * 16, 16)] = v
            c1 = pltpu.async_copy(p_hbm.at[dit],
                                  prs[0].at[pl.ds(0, max(tail_p, 16))],
                                  gsems[0])
            c2 = pltpu.async_copy(q_hbm.at[sit],
                                  qrs[0].at[pl.ds(0, max(tail_p, 16))],
                                  gsems[0])
            c1.wait()
            c2.wait()
            combine(prs[0], qrs[0], hbs[0], tail)
            pltpu.sync_copy(hbs[0].at[pl.ds(0, tail)],
                            hp_hbm.at[pl.ds(off, tail)])

    return k(p_tab, q_tab, d_idx, s_idx)


def _sc_scatter(msg, d_idx, zeros, n_half):
    e = d_idx.shape[0]
    half = n_half // SC_CORES
    pad = zeros.shape[0]
    rows_pt = pad // SC_SUBCORES
    chunk = 112
    per_t = e // SC_SUBCORES
    iters = per_t // chunk
    tail = per_t - iters * chunk
    depth = 2
    supers = iters // depth
    rem = iters - supers * depth
    mesh = plsc.VectorSubcoreMesh(core_axis_name="c", subcore_axis_name="s")

    @functools.partial(
        pl.kernel, mesh=mesh,
        out_type=jax.ShapeDtypeStruct((SC_CORES, pad, PK), F32),
        scratch_types=(
            [pltpu.VMEM((chunk,), jnp.int32)] * depth
            + [pltpu.VMEM((chunk, PK), F32)] * depth
            + [pltpu.SemaphoreType.DMA] * (2 * depth)
            + [pltpu.VMEM((max((tail + 15) // 16 * 16, 16),), jnp.int32),
               pltpu.VMEM_SHARED((pad, PK), F32)]
        ),
    )
    def k(msg_hbm, d_hbm, z_hbm, out_hbm, *refs):
        draws = refs[:depth]
        mbufs = refs[depth:2 * depth]
        lsems = refs[2 * depth:3 * depth]
        ssems = refs[3 * depth:4 * depth]
        drawt, acc = refs[4 * depth:]
        cid = lax.axis_index("c")
        sid = lax.axis_index("s")
        base = cid * half
        dump = half + sid

        zr = sid * rows_pt
        pltpu.sync_copy(z_hbm.at[pl.ds(zr, rows_pt)],
                        acc.at[pl.ds(zr, rows_pt)])
        plsc.subcore_barrier()

        def localize(ref, count):
            for kk in range((count + 15) // 16):
                v = ref[pl.ds(kk * 16, 16)]
                v = jnp.where(v >= n_half, v - n_half, v)
                l = v - base
                ok = (l >= 0) & (l < half)
                if (kk + 1) * 16 > count:
                    lane = lax.iota(jnp.int32, 16)
                    ok = ok & (lane < count - kk * 16)
                ref[pl.ds(kk * 16, 16)] = jnp.where(ok, l, dump)

        def pipe_group(off0):
            loads = []
            for b in range(depth):
                off = off0 + b * chunk
                c1 = pltpu.async_copy(d_hbm.at[pl.ds(off, chunk)],
                                      draws[b], lsems[b])
                c2 = pltpu.async_copy(msg_hbm.at[pl.ds(off, chunk)],
                                      mbufs[b], lsems[b])
                loads.append((c1, c2))
            scats = []
            for b in range(depth):
                loads[b][0].wait()
                loads[b][1].wait()
                localize(draws[b], chunk)
                scats.append(pltpu.async_copy(mbufs[b], acc.at[draws[b]],
                                              ssems[b], add=True))
            for s in scats:
                s.wait()

        def body(j, carry):
            pipe_group(sid * per_t + j * (depth * chunk))
            return carry

        lax.fori_loop(0, supers, body, 0)
        for r in range(rem):
            off = sid * per_t + (supers * depth + r) * chunk
            pltpu.sync_copy(d_hbm.at[pl.ds(off, chunk)], draws[0])
            pltpu.sync_copy(msg_hbm.at[pl.ds(off, chunk)], mbufs[0])
            localize(draws[0], chunk)
            pltpu.sync_copy(mbufs[0], acc.at[draws[0]], add=True)
        if tail:
            tail_p = (tail + 15) // 16 * 16
            off = sid * per_t + iters * chunk
            pltpu.sync_copy(d_hbm.at[pl.ds(off, tail)],
                            drawt.at[pl.ds(0, tail)])
            pltpu.sync_copy(msg_hbm.at[pl.ds(off, tail)],
                            mbufs[0].at[pl.ds(0, tail)])
            localize(drawt, tail)
            pltpu.sync_copy(mbufs[0].at[pl.ds(0, tail_p)],
                            acc.at[drawt], add=True)

        plsc.subcore_barrier()
        pltpu.sync_copy(acc.at[pl.ds(zr, rows_pt)],
                        out_hbm.at[cid, pl.ds(zr, rows_pt)])

    out = k(msg, d_idx, zeros)
    return jnp.concatenate([out[0, :half], out[1, :half]], axis=0)


def _conv(cp, left, right, s_idx, d_idx, s2, d2, edge_b, zeros):
    n_half = right.shape[0]
    e = d_idx.shape[0]
    step = ((e // 4) // 32000 * 32000) or ((e // 2) or e)
    bounds = []
    o = 0
    while o + step < e:
        bounds.append((o, step))
        o += step
        if len(bounds) == 3:
            break
    bounds.append((o, e - o))
    blc = (cp['bl'] + edge_b * cp['We'][:, 0]).reshape(1, EMBD)
    p_tab, q_tab = _pq(right, left, cp['Wl'], blc, cp['Wr'])
    aggs = []
    for o, sz in bounds:
        hpre = _sc_gather(p_tab, q_tab, d_idx[o:o + sz], s_idx[o:o + sz])
        msg = _edge_mlp(hpre, d2[o:o + sz], cp['ln_g'],
                        cp['ln_b'], cp['Wf'], cp['bf'], n_half)
        aggs.append(_sc_scatter(msg, d_idx[o:o + sz], zeros, n_half))
    return _post(aggs, right, cp['pc_g'], cp['pc_b'],
                 cp['Wo1'][:, :EMBD], cp['Wo1'][:, EMBD:], cp['bo1'],
                 cp['Wo2'], cp['bo2'])


def kernel(params, constraint_features, edge_indices, edge_features,
           variable_features):
    p = params
    src = edge_indices[0]
    dst = edge_indices[1]
    e = src.shape[0]
    src2 = src.reshape(e, 1)
    dst2 = dst.reshape(e, 1)
    n_c = constraint_features.shape[0]
    edge_b = p['edge_ln_b'][0]

    half = n_c // 2 // SC_CORES
    pad = ((half + SC_SUBCORES) + 127) // 128 * 128
    zeros = jnp.zeros((pad, PK), F32)

    cf, cs = _embed(constraint_features, p['cons_ln_g'], p['cons_ln_b'],
                    p['cons_W1'], p['cons_b1'], p['cons_W2'], p['cons_b2'])
    cf = _se_apply(cf, cs, p['se_con_W1'], p['se_con_W2'])
    vf, vs = _embed(variable_features, p['var_ln_g'], p['var_ln_b'],
                    p['var_W1'], p['var_b1'], p['var_W2'], p['var_b2'])
    vf = _se_apply(vf, vs, p['se_var_W1'], p['se_var_W2'])

    vf = _conv(p['c2v'], cf, vf, src, dst, src2, dst2, edge_b, zeros)
    cf = _conv(p['v2c'], vf, cf, dst, src, dst2, src2, edge_b, zeros)
    vf = _conv(p['c2v2'], cf, vf, src, dst, src2, dst2, edge_b, zeros)
    cf = _conv(p['v2c2'], vf, cf, dst, src, dst2, src2, edge_b, zeros)

    con_out = _head(cf, p['con_W1'], p['con_b1'], p['con_W2'])
    var_out = _head(vf, p['varm_W1'], p['varm_b1'], p['varm_W2'])
    return jnp.squeeze(con_out, -1), jnp.squeeze(var_out, -1)
